# Initial kernel scaffold; baseline (speedup 1.0000x reference)
import jax
import jax.numpy as jnp
from jax.experimental import pallas as pl


def kernel(x, xyz, w_fc1, b_fc1, bn1_g, bn1_b, mem_decay, thr_adapt, refr_decay, thr_base, w_q, w_k, w_v, w_d1, b_d1, w_d2, b_d2, w_g1, b_g1, w_g2, b_g2, w_fc2, b_fc2, bn2_g, bn2_b):
    raise NotImplementedError("write your pallas kernel here")


# speedup vs baseline: 9.1555x; 9.1555x over previous
import functools

import jax
import jax.numpy as jnp
from jax import lax
from jax.experimental import pallas as pl
from jax.experimental.pallas import tpu as pltpu
from jax.experimental.pallas import tpu_sc as plsc

B, N, DP, DM, K, H, T = 4, 2048, 128, 256, 16, 4, 4
HD = DM // H
INV_SQRT_2PI = 0.3989422804014327
BN_SCALE = 1.0 / (1.0 + 1e-5) ** 0.5

PA = 256


def _knn_body(xt_ref, xf_ref, idx_ref):
    b = pl.program_id(0)
    xt = xt_ref[0]
    xf = xf_ref[0]
    s = lax.dot_general(2.0 * xt, xf, (((1,), (1,)), ((), ())),
                        preferred_element_type=jnp.float32)
    ones = jnp.ones((1, 8), jnp.float32)
    xx = lax.dot_general(ones, xf * xf, (((1,), (1,)), ((), ())),
                         preferred_element_type=jnp.float32)
    s = s - xx
    iota = lax.broadcasted_iota(jnp.int32, (PA, N), 1)
    cols = []
    for _ in range(K):
        m = jnp.max(s, axis=1, keepdims=True)
        cand = jnp.where(s >= m, iota, N)
        a = jnp.min(cand, axis=1, keepdims=True)
        s = jnp.where(cand == a, -3.0e38, s)
        cols.append(a)
    idx_ref[0] = jnp.concatenate(cols, axis=1) + b * N


def _knn_topk(xyzp):
    return pl.pallas_call(
        _knn_body,
        grid=(B, N // PA),
        in_specs=[
            pl.BlockSpec((1, PA, 8), lambda b, n: (b, n, 0)),
            pl.BlockSpec((1, N, 8), lambda b, n: (b, 0, 0)),
        ],
        out_specs=pl.BlockSpec((1, PA, K), lambda b, n: (b, n, 0)),
        out_shape=jax.ShapeDtypeStruct((B, N, K), jnp.int32),
    )(xyzp, xyzp)


PB = 256


def _sigmoid(v):
    return 1.0 / (1.0 + jnp.exp(-v))


def _spike_fn(v):
    vc = jnp.clip(v, -10.0, 10.0)
    return 0.5 * jnp.exp(-vc * vc * 0.5) * INV_SQRT_2PI + 0.5 * _sigmoid(10.0 * vc)


def _lif_rows(x, mdv, tav, rdv, tb):
    mem = jnp.zeros_like(x)
    thr = jnp.broadcast_to(tb, x.shape)
    refr = jnp.zeros_like(x)
    acc = jnp.zeros_like(x)
    for _ in range(T):
        xi = x * (refr <= 0.0).astype(x.dtype)
        mem = mem * mdv * (1.0 - refr) + xi
        sp = _spike_fn(mem - thr)
        mem = mem * (1.0 - sp)
        refr = refr * rdv + sp
        thr = thr + tav * sp
        thr = tb + (thr - tb) * 0.95
        acc = acc + sp
    return acc * (1.0 / T)


def _feat_body(x_ref, wfc1_ref, alpha_ref, beta_ref, md_ref, ta_ref, rd_ref,
               tb_ref, wq_ref, wk_ref, wv_ref, q_ref, kv_ref):
    xb = x_ref[0]
    pre = lax.dot_general(xb, wfc1_ref[...], (((0,), (1,)), ((), ())),
                          preferred_element_type=jnp.float32)
    pre = pre * alpha_ref[...] + beta_ref[...]
    mdv = jnp.clip(md_ref[...], 0.1, 0.99)
    tav = jnp.clip(ta_ref[...], 0.001, 0.1)
    rdv = jnp.clip(rd_ref[...], 0.1, 0.95)
    f = _lif_rows(pre, mdv, tav, rdv, tb_ref[...])
    nt = (((1,), (1,)), ((), ()))
    q_ref[0] = lax.dot_general(f, wq_ref[...], nt,
                               preferred_element_type=jnp.float32)
    kv_ref[0, :, :DM] = lax.dot_general(f, wk_ref[...], nt,
                                        preferred_element_type=jnp.float32)
    kv_ref[0, :, DM:] = lax.dot_general(f, wv_ref[...], nt,
                                        preferred_element_type=jnp.float32)


def _features(x, w_fc1, alpha1, beta1, md, ta, rd, tb, w_q, w_k, w_v):
    full = lambda a: pl.BlockSpec(a.shape, lambda b, n: (0,) * a.ndim)
    return pl.pallas_call(
        _feat_body,
        grid=(B, N // PB),
        in_specs=[
            pl.BlockSpec((1, DP, PB), lambda b, n: (b, 0, n)),
            full(w_fc1), full(alpha1), full(beta1), full(md), full(ta),
            full(rd), full(tb), full(w_q), full(w_k), full(w_v),
        ],
        out_specs=[
            pl.BlockSpec((1, PB, DM), lambda b, n: (b, n, 0)),
            pl.BlockSpec((1, PB, 2 * DM), lambda b, n: (b, n, 0)),
        ],
        out_shape=[
            jax.ShapeDtypeStruct((B, N, DM), jnp.float32),
            jax.ShapeDtypeStruct((B, N, 2 * DM), jnp.float32),
        ],
    )(x, w_fc1, alpha1, beta1, md, ta, rd, tb, w_q, w_k, w_v)


NW = 32
CH = 128


def _sc_gather_kv(kv_flat, idxg):
    rows = idxg.shape[0]
    rpw = rows // NW
    nch = rpw // CH
    mesh = plsc.VectorSubcoreMesh(core_axis_name="c", subcore_axis_name="s")

    @functools.partial(
        pl.kernel, mesh=mesh,
        out_type=jax.ShapeDtypeStruct((rows, 2 * DM), jnp.float32),
        scratch_types=[
            pltpu.VMEM((CH,), jnp.int32),
            pltpu.VMEM((CH, 2 * DM), jnp.float32),
            pltpu.SemaphoreType.DMA,
        ],
    )
    def gather(kv_hbm, idx_hbm, kvn_hbm, idx_v, kv_v, sem1):
        wid = lax.axis_index("s") * 2 + lax.axis_index("c")
        base = wid * rpw

        def body(i, carry):
            off = base + i * CH
            pltpu.sync_copy(idx_hbm.at[pl.ds(off, CH)], idx_v)
            pltpu.async_copy(kv_hbm.at[idx_v], kv_v, sem1).wait()
            pltpu.sync_copy(kv_v, kvn_hbm.at[pl.ds(off, CH)])
            return carry

        lax.fori_loop(0, nch, body, 0)

    return gather(kv_flat, idxg)


def _sc_gather_xyz(xyzp_vec, idxg):
    rows = idxg.shape[0]
    rpw = rows // NW
    nch = rpw // CH
    mesh = plsc.VectorSubcoreMesh(core_axis_name="c", subcore_axis_name="s")

    @functools.partial(
        pl.kernel, mesh=mesh,
        out_type=jax.ShapeDtypeStruct((rows * 8,), jnp.float32),
        scratch_types=[
            pltpu.VMEM((B * N * 8,), jnp.float32),
            pltpu.VMEM((CH,), jnp.int32),
            pltpu.VMEM((CH * 8,), jnp.float32),
        ],
        compiler_params=pltpu.CompilerParams(needs_layout_passes=False),
    )
    def gather(xyzp_hbm, idx_hbm, xyzn_hbm, tab_v, idx_v, out_v):
        wid = lax.axis_index("s") * 2 + lax.axis_index("c")
        base = wid * rpw
        pltpu.sync_copy(xyzp_hbm, tab_v)
        lane8 = lax.iota(jnp.int32, 16) * 8

        def body(i, carry):
            off = base + i * CH
            pltpu.sync_copy(idx_hbm.at[pl.ds(off, CH)], idx_v)
            for jj in range(CH // 16):
                rbase = idx_v[pl.ds(jj * 16, 16)] * 8
                for c in range(8):
                    vals = plsc.load_gather(tab_v, [rbase + c])
                    plsc.store_scatter(out_v, [lane8 + (jj * 128 + c)], vals)
            pltpu.sync_copy(out_v, xyzn_hbm.at[pl.ds(off * 8, CH * 8)])
            return carry

        lax.fori_loop(0, nch, body, 0)

    return gather(xyzp_vec, idxg)


def _sc_gather(kv_flat, xyzp_flat, idxg):
    kvn = _sc_gather_kv(kv_flat, idxg)
    xyzn = _sc_gather_xyz(xyzp_flat.reshape(-1), idxg)
    return kvn, xyzn.reshape(-1, 8)


PD = 128
S = PD * K


def _attn_body(q_ref, kvn_ref, xyzn_ref, xq_ref, x_ref, wd1_ref, bd1_ref,
               wd2_ref, bd2_ref, wg1_ref, bg1_ref, wg2_ref, bg2_ref,
               wfc2_ref, beta2_ref, out_ref):
    kv = kvn_ref[0]
    kn = kv[:, :DM]
    vn = kv[:, DM:]
    xq = xq_ref[0]
    rel = jnp.repeat(xq, K, axis=0) - xyzn_ref[0]
    nt = (((1,), (1,)), ((), ()))
    h1 = lax.dot_general(rel, wd1_ref[...], nt,
                         preferred_element_type=jnp.float32)
    h1 = jnp.maximum(h1 + bd1_ref[...], 0.0)

    q = q_ref[0]
    qe = jnp.repeat(q, K, axis=0)
    prod = (qe * kn).reshape(S, H, HD)
    logits = jnp.sum(prod, axis=2) * (1.0 / (HD ** 0.5))
    lo = logits.reshape(PD, K, H)
    m = lo[:, 0, :]
    for k in range(1, K):
        m = jnp.maximum(m, lo[:, k, :])
    es = []
    ssum = jnp.zeros((PD, H), jnp.float32)
    for k in range(K):
        e = jnp.exp(lo[:, k, :] - m)
        es.append(e)
        ssum = ssum + e
    inv = 1.0 / ssum
    vn3 = vn.reshape(PD, K, DM)
    h13 = h1.reshape(PD, K, DM)
    vsum = jnp.zeros((PD, DM), jnp.float32)
    asum = jnp.zeros((PD, DM), jnp.float32)
    for k in range(K):
        w = (es[k] * inv)[:, :, None]
        wf = jnp.broadcast_to(w, (PD, H, HD)).reshape(PD, DM)
        vsum = vsum + wf * vn3[:, k, :]
        asum = asum + wf * h13[:, k, :]
    attn_out = vsum + lax.dot_general(asum, wd2_ref[...], nt,
                                      preferred_element_type=jnp.float32)
    attn_out = attn_out + bd2_ref[...]

    g = lax.dot_general(attn_out, wg1_ref[...], nt,
                        preferred_element_type=jnp.float32)
    g = jnp.maximum(g + bg1_ref[...], 0.0)
    g = lax.dot_general(g, wg2_ref[...], nt,
                        preferred_element_type=jnp.float32) + bg2_ref[...]
    res = lax.dot_general(wfc2_ref[...], g, nt,
                          preferred_element_type=jnp.float32)
    out_ref[0] = res + beta2_ref[...] + x_ref[0]


def _attention(q, kvn, xyzn, xyzp, x, wd1p, b_d1, w_d2, b_d2, w_g1, b_g1,
               w_g2, b_g2, wfc2s, beta2):
    full = lambda a: pl.BlockSpec(a.shape, lambda b, n: (0,) * a.ndim)
    return pl.pallas_call(
        _attn_body,
        grid=(B, N // PD),
        in_specs=[
            pl.BlockSpec((1, PD, DM), lambda b, n: (b, n, 0)),
            pl.BlockSpec((1, S, 2 * DM), lambda b, n: (b, n, 0)),
            pl.BlockSpec((1, S, 8), lambda b, n: (b, n, 0)),
            pl.BlockSpec((1, PD, 8), lambda b, n: (b, n, 0)),
            pl.BlockSpec((1, DP, PD), lambda b, n: (b, 0, n)),
            full(wd1p), full(b_d1), full(w_d2), full(b_d2), full(w_g1),
            full(b_g1), full(w_g2), full(b_g2), full(wfc2s), full(beta2),
        ],
        out_specs=pl.BlockSpec((1, DP, PD), lambda b, n: (b, 0, n)),
        out_shape=jax.ShapeDtypeStruct((B, DP, N), jnp.float32),
    )(q, kvn, xyzn, xyzp, x, wd1p, b_d1, w_d2, b_d2, w_g1, b_g1, w_g2, b_g2,
      wfc2s, beta2)


def kernel(x, xyz, w_fc1, b_fc1, bn1_g, bn1_b, mem_decay, thr_adapt,
           refr_decay, thr_base, w_q, w_k, w_v, w_d1, b_d1, w_d2, b_d2,
           w_g1, b_g1, w_g2, b_g2, w_fc2, b_fc2, bn2_g, bn2_b):
    xyzp = jnp.concatenate(
        [xyz, jnp.zeros((B, N, 5), jnp.float32)], axis=-1)

    idx = _knn_topk(xyzp)

    alpha1 = (bn1_g * BN_SCALE).reshape(1, DM)
    beta1 = (b_fc1 * bn1_g * BN_SCALE + bn1_b).reshape(1, DM)
    q, kv = _features(x, w_fc1, alpha1, beta1,
                      mem_decay.reshape(1, DM), thr_adapt.reshape(1, DM),
                      refr_decay.reshape(1, DM), thr_base.reshape(1, DM),
                      w_q, w_k, w_v)

    kvn_flat, xyzn_flat = _sc_gather(
        kv.reshape(B * N, 2 * DM), xyzp.reshape(B * N, 8),
        idx.reshape(B * N * K))

    wd1p = jnp.concatenate([w_d1, jnp.zeros((DM, 5), jnp.float32)], axis=-1)
    alpha2 = bn2_g * BN_SCALE
    wfc2s = w_fc2 * alpha2[:, None]
    beta2 = (b_fc2 * alpha2 + bn2_b).reshape(DP, 1)

    return _attention(q, kvn_flat.reshape(B, N * K, 2 * DM),
                      xyzn_flat.reshape(B, N * K, 8), xyzp, x,
                      wd1p, b_d1.reshape(1, DM), w_d2, b_d2.reshape(1, DM),
                      w_g1, b_g1.reshape(1, DM), w_g2, b_g2.reshape(1, DM),
                      wfc2s, beta2)


# speedup vs baseline: 13.2039x; 1.4422x over previous
import functools

import jax
import jax.numpy as jnp
from jax import lax
from jax.experimental import pallas as pl
from jax.experimental.pallas import tpu as pltpu
from jax.experimental.pallas import tpu_sc as plsc

B, N, DP, DM, K, H, T = 4, 2048, 128, 256, 16, 4, 4
HD = DM // H
INV_SQRT_2PI = 0.3989422804014327
BN_SCALE = 1.0 / (1.0 + 1e-5) ** 0.5

PA = 256


def _knn_body(xt_ref, xf_ref, idx_ref):
    b = pl.program_id(0)
    xt = xt_ref[0]
    xf = xf_ref[0]
    s = lax.dot_general(2.0 * xt, xf, (((1,), (1,)), ((), ())),
                        preferred_element_type=jnp.float32)
    ones = jnp.ones((1, 8), jnp.float32)
    xx = lax.dot_general(ones, xf * xf, (((1,), (1,)), ((), ())),
                         preferred_element_type=jnp.float32)
    s = s - xx
    iota = lax.broadcasted_iota(jnp.int32, (PA, N), 1)
    cols = []
    for _ in range(K):
        m = jnp.max(s, axis=1, keepdims=True)
        msk = s >= m
        cand = jnp.where(msk, iota, N)
        a = jnp.min(cand, axis=1, keepdims=True)
        s = jnp.where(msk, -3.0e38, s)
        cols.append(a)
    idx_ref[0] = jnp.concatenate(cols, axis=1) + b * N


def _knn_topk(xyzp):
    return pl.pallas_call(
        _knn_body,
        grid=(B, N // PA),
        in_specs=[
            pl.BlockSpec((1, PA, 8), lambda b, n: (b, n, 0)),
            pl.BlockSpec((1, N, 8), lambda b, n: (b, 0, 0)),
        ],
        out_specs=pl.BlockSpec((1, PA, K), lambda b, n: (b, n, 0)),
        out_shape=jax.ShapeDtypeStruct((B, N, K), jnp.int32),
    )(xyzp, xyzp)


PB = 256


def _sigmoid(v):
    return 1.0 / (1.0 + jnp.exp(-v))


def _spike_fn(v):
    vc = jnp.clip(v, -10.0, 10.0)
    return 0.5 * jnp.exp(-vc * vc * 0.5) * INV_SQRT_2PI + 0.5 * _sigmoid(10.0 * vc)


def _lif_rows(x, mdv, tav, rdv, tb):
    mem = jnp.zeros_like(x)
    thr = jnp.broadcast_to(tb, x.shape)
    refr = jnp.zeros_like(x)
    acc = jnp.zeros_like(x)
    for _ in range(T):
        xi = x * (refr <= 0.0).astype(x.dtype)
        mem = mem * mdv * (1.0 - refr) + xi
        sp = _spike_fn(mem - thr)
        mem = mem * (1.0 - sp)
        refr = refr * rdv + sp
        thr = thr + tav * sp
        thr = tb + (thr - tb) * 0.95
        acc = acc + sp
    return acc * (1.0 / T)


def _feat_body(x_ref, wfc1_ref, alpha_ref, beta_ref, md_ref, ta_ref, rd_ref,
               tb_ref, wq_ref, wk_ref, wv_ref, q_ref, kv_ref):
    xb = x_ref[0]
    pre = lax.dot_general(xb, wfc1_ref[...], (((0,), (1,)), ((), ())),
                          preferred_element_type=jnp.float32)
    pre = pre * alpha_ref[...] + beta_ref[...]
    mdv = jnp.clip(md_ref[...], 0.1, 0.99)
    tav = jnp.clip(ta_ref[...], 0.001, 0.1)
    rdv = jnp.clip(rd_ref[...], 0.1, 0.95)
    f = _lif_rows(pre, mdv, tav, rdv, tb_ref[...])
    nt = (((1,), (1,)), ((), ()))
    q_ref[0] = lax.dot_general(f, wq_ref[...], nt,
                               preferred_element_type=jnp.float32)
    kv_ref[0, :, :DM] = lax.dot_general(f, wk_ref[...], nt,
                                        preferred_element_type=jnp.float32)
    kv_ref[0, :, DM:] = lax.dot_general(f, wv_ref[...], nt,
                                        preferred_element_type=jnp.float32)


def _features(x, w_fc1, alpha1, beta1, md, ta, rd, tb, w_q, w_k, w_v):
    full = lambda a: pl.BlockSpec(a.shape, lambda b, n: (0,) * a.ndim)
    return pl.pallas_call(
        _feat_body,
        grid=(B, N // PB),
        in_specs=[
            pl.BlockSpec((1, DP, PB), lambda b, n: (b, 0, n)),
            full(w_fc1), full(alpha1), full(beta1), full(md), full(ta),
            full(rd), full(tb), full(w_q), full(w_k), full(w_v),
        ],
        out_specs=[
            pl.BlockSpec((1, PB, DM), lambda b, n: (b, n, 0)),
            pl.BlockSpec((1, PB, 2 * DM), lambda b, n: (b, n, 0)),
        ],
        out_shape=[
            jax.ShapeDtypeStruct((B, N, DM), jnp.float32),
            jax.ShapeDtypeStruct((B, N, 2 * DM), jnp.float32),
        ],
    )(x, w_fc1, alpha1, beta1, md, ta, rd, tb, w_q, w_k, w_v)


NW = 32
CH = 128


def _sc_gather_kv(kv_flat, idxg):
    rows = idxg.shape[0]
    rpw = rows // NW
    nch = rpw // CH
    mesh = plsc.VectorSubcoreMesh(core_axis_name="c", subcore_axis_name="s")

    @functools.partial(
        pl.kernel, mesh=mesh,
        out_type=jax.ShapeDtypeStruct((rows, 2 * DM), jnp.float32),
        scratch_types=[
            pltpu.VMEM((CH,), jnp.int32),
            pltpu.VMEM((CH, 2 * DM), jnp.float32),
            pltpu.SemaphoreType.DMA,
        ],
    )
    def gather(kv_hbm, idx_hbm, kvn_hbm, idx_v, kv_v, sem1):
        wid = lax.axis_index("s") * 2 + lax.axis_index("c")
        base = wid * rpw

        def body(i, carry):
            off = base + i * CH
            pltpu.sync_copy(idx_hbm.at[pl.ds(off, CH)], idx_v)
            pltpu.async_copy(kv_hbm.at[idx_v], kv_v, sem1).wait()
            pltpu.sync_copy(kv_v, kvn_hbm.at[pl.ds(off, CH)])
            return carry

        lax.fori_loop(0, nch, body, 0)

    return gather(kv_flat, idxg)


def _sc_gather_xyz(xyzp_vec, idxg):
    rows = idxg.shape[0]
    rpw = rows // NW
    nch = rpw // CH
    mesh = plsc.VectorSubcoreMesh(core_axis_name="c", subcore_axis_name="s")

    @functools.partial(
        pl.kernel, mesh=mesh,
        out_type=jax.ShapeDtypeStruct((rows * 8,), jnp.float32),
        scratch_types=[
            pltpu.VMEM((B * N * 8,), jnp.float32),
            pltpu.VMEM((CH,), jnp.int32),
            pltpu.VMEM((CH * 8,), jnp.float32),
        ],
        compiler_params=pltpu.CompilerParams(needs_layout_passes=False),
    )
    def gather(xyzp_hbm, idx_hbm, xyzn_hbm, tab_v, idx_v, out_v):
        wid = lax.axis_index("s") * 2 + lax.axis_index("c")
        base = wid * rpw
        pltpu.sync_copy(xyzp_hbm, tab_v)
        lane8 = lax.iota(jnp.int32, 16) * 8

        def body(i, carry):
            off = base + i * CH
            pltpu.sync_copy(idx_hbm.at[pl.ds(off, CH)], idx_v)
            for jj in range(CH // 16):
                rbase = idx_v[pl.ds(jj * 16, 16)] * 8
                for c in range(8):
                    vals = plsc.load_gather(tab_v, [rbase + c])
                    plsc.store_scatter(out_v, [lane8 + (jj * 128 + c)], vals)
            pltpu.sync_copy(out_v, xyzn_hbm.at[pl.ds(off * 8, CH * 8)])
            return carry

        lax.fori_loop(0, nch, body, 0)

    return gather(xyzp_vec, idxg)


def _sc_gather(kv_flat, xyzp_flat, idxg):
    kvn = _sc_gather_kv(kv_flat, idxg)
    xyzn = _sc_gather_xyz(xyzp_flat.reshape(-1), idxg)
    return kvn, xyzn.reshape(-1, 8)


PD = 128
S = PD * K


def _attn_body(q_ref, kvn_ref, xyzn_ref, xq_ref, x_ref, gcat_ref, bcat_ref,
               wd1c_ref, bd1t_ref, wd2_ref, bd2_ref, wg1_ref, bg1_ref,
               wg2_ref, bg2_ref, wfc2_ref, beta2_ref, out_ref):
    kv4 = kvn_ref[0]
    xyzn4 = xyzn_ref[0]
    xq = xq_ref[0]
    q = q_ref[0]
    nt = (((1,), (1,)), ((), ()))
    nn = (((1,), (0,)), ((), ()))

    rel_cat = jnp.concatenate([xq - xyzn4[k] for k in range(K)], axis=1)
    h1_cat = lax.dot_general(rel_cat, wd1c_ref[...], nn,
                             preferred_element_type=jnp.float32)
    h1_cat = jnp.maximum(h1_cat + bd1t_ref[...], 0.0)

    prod_cat = jnp.concatenate([q * kv4[k, :, :DM] for k in range(K)], axis=1)
    logits = lax.dot_general(prod_cat, gcat_ref[...], nn,
                             preferred_element_type=jnp.float32)
    l3 = (logits[:, :K * H] * (1.0 / (HD ** 0.5))).reshape(PD, K, H)
    m = jnp.max(l3, axis=1, keepdims=True)
    e = jnp.exp(l3 - m)
    ssum = jnp.sum(e, axis=1, keepdims=True)
    attn = (e / ssum).reshape(PD, K * H)
    wf_cat = lax.dot_general(attn, bcat_ref[...], nn,
                             preferred_element_type=jnp.float32)

    vsum = jnp.zeros((PD, DM), jnp.float32)
    asum = jnp.zeros((PD, DM), jnp.float32)
    for k in range(K):
        wf = wf_cat[:, k * DM:(k + 1) * DM]
        vsum = vsum + wf * kv4[k, :, DM:]
        asum = asum + wf * h1_cat[:, k * DM:(k + 1) * DM]
    attn_out = vsum + lax.dot_general(asum, wd2_ref[...], nt,
                                      preferred_element_type=jnp.float32)
    attn_out = attn_out + bd2_ref[...]

    g = lax.dot_general(attn_out, wg1_ref[...], nt,
                        preferred_element_type=jnp.float32)
    g = jnp.maximum(g + bg1_ref[...], 0.0)
    g = lax.dot_general(g, wg2_ref[...], nt,
                        preferred_element_type=jnp.float32) + bg2_ref[...]
    res = lax.dot_general(wfc2_ref[...], g, nt,
                          preferred_element_type=jnp.float32)
    out_ref[0] = res + beta2_ref[...] + x_ref[0]


def _attention(q, kvn, xyzn, xyzp, x, gcat, bcat, wd1c, bd1t, w_d2, b_d2,
               w_g1, b_g1, w_g2, b_g2, wfc2s, beta2):
    full = lambda a: pl.BlockSpec(a.shape, lambda b, n: (0,) * a.ndim)
    return pl.pallas_call(
        _attn_body,
        grid=(B, N // PD),
        in_specs=[
            pl.BlockSpec((1, PD, DM), lambda b, n: (b, n, 0)),
            pl.BlockSpec((1, K, PD, 2 * DM), lambda b, n: (b, 0, n, 0)),
            pl.BlockSpec((1, K, PD, 8), lambda b, n: (b, 0, n, 0)),
            pl.BlockSpec((1, PD, 8), lambda b, n: (b, n, 0)),
            pl.BlockSpec((1, DP, PD), lambda b, n: (b, 0, n)),
            full(gcat), full(bcat), full(wd1c), full(bd1t), full(w_d2),
            full(b_d2), full(w_g1), full(b_g1), full(w_g2), full(b_g2),
            full(wfc2s), full(beta2),
        ],
        out_specs=pl.BlockSpec((1, DP, PD), lambda b, n: (b, 0, n)),
        out_shape=jax.ShapeDtypeStruct((B, DP, N), jnp.float32),
    )(q, kvn, xyzn, xyzp, x, gcat, bcat, wd1c, bd1t, w_d2, b_d2, w_g1, b_g1,
      w_g2, b_g2, wfc2s, beta2)


def kernel(x, xyz, w_fc1, b_fc1, bn1_g, bn1_b, mem_decay, thr_adapt,
           refr_decay, thr_base, w_q, w_k, w_v, w_d1, b_d1, w_d2, b_d2,
           w_g1, b_g1, w_g2, b_g2, w_fc2, b_fc2, bn2_g, bn2_b):
    xyzp = jnp.concatenate(
        [xyz, jnp.zeros((B, N, 5), jnp.float32)], axis=-1)

    idx = _knn_topk(xyzp)

    alpha1 = (bn1_g * BN_SCALE).reshape(1, DM)
    beta1 = (b_fc1 * bn1_g * BN_SCALE + bn1_b).reshape(1, DM)
    q, kv = _features(x, w_fc1, alpha1, beta1,
                      mem_decay.reshape(1, DM), thr_adapt.reshape(1, DM),
                      refr_decay.reshape(1, DM), thr_base.reshape(1, DM),
                      w_q, w_k, w_v)

    idxg = idx.transpose(0, 2, 1).reshape(B * N * K)
    kvn_flat, xyzn_flat = _sc_gather(
        kv.reshape(B * N, 2 * DM), xyzp.reshape(B * N, 8), idxg)

    wd1p = jnp.concatenate([w_d1, jnp.zeros((DM, 5), jnp.float32)], axis=-1)
    wd1c = jnp.kron(jnp.eye(K, dtype=jnp.float32), wd1p.T)
    bd1t = jnp.tile(b_d1.reshape(1, DM), (1, K))
    h_of_m = jnp.arange(DM, dtype=jnp.int32) // HD
    g3 = jax.nn.one_hot(H * jnp.arange(K, dtype=jnp.int32)[:, None]
                        + h_of_m[None, :], 128, dtype=jnp.float32)
    gcat = g3.reshape(K * DM, 128)
    bcat = gcat[:, :K * H].T

    alpha2 = bn2_g * BN_SCALE
    wfc2s = w_fc2 * alpha2[:, None]
    beta2 = (b_fc2 * alpha2 + bn2_b).reshape(DP, 1)

    return _attention(q, kvn_flat.reshape(B, K, N, 2 * DM),
                      xyzn_flat.reshape(B, K, N, 8), xyzp, x,
                      gcat, bcat, wd1c, bd1t, w_d2, b_d2.reshape(1, DM),
                      w_g1, b_g1.reshape(1, DM), w_g2, b_g2.reshape(1, DM),
                      wfc2s, beta2)


# speedup vs baseline: 14.9468x; 1.1320x over previous
import functools

import jax
import jax.numpy as jnp
from jax import lax
from jax.experimental import pallas as pl
from jax.experimental.pallas import tpu as pltpu
from jax.experimental.pallas import tpu_sc as plsc

B, N, DP, DM, K, H, T = 4, 2048, 128, 256, 16, 4, 4
HD = DM // H
INV_SQRT_2PI = 0.3989422804014327
BN_SCALE = 1.0 / (1.0 + 1e-5) ** 0.5

PA = 256


def _knn_body(xt_ref, xf_ref, idx_ref):
    b = pl.program_id(0)
    xt = xt_ref[0]
    xf = xf_ref[0]
    s = lax.dot_general(2.0 * xt, xf, (((1,), (1,)), ((), ())),
                        preferred_element_type=jnp.float32)
    ones = jnp.ones((1, 8), jnp.float32)
    xx = lax.dot_general(ones, xf * xf, (((1,), (1,)), ((), ())),
                         preferred_element_type=jnp.float32)
    s = s - xx
    iota = lax.broadcasted_iota(jnp.int32, (PA, N), 1)
    cols = []
    for _ in range(K):
        m = jnp.max(s, axis=1, keepdims=True)
        msk = s >= m
        cand = jnp.where(msk, iota, N)
        a = jnp.min(cand, axis=1, keepdims=True)
        s = jnp.where(msk, -3.0e38, s)
        cols.append(a)
    idx_ref[0] = jnp.concatenate(cols, axis=1) + b * N


def _knn_topk(xyzp):
    return pl.pallas_call(
        _knn_body,
        grid=(B, N // PA),
        in_specs=[
            pl.BlockSpec((1, PA, 8), lambda b, n: (b, n, 0)),
            pl.BlockSpec((1, N, 8), lambda b, n: (b, 0, 0)),
        ],
        out_specs=pl.BlockSpec((1, PA, K), lambda b, n: (b, n, 0)),
        out_shape=jax.ShapeDtypeStruct((B, N, K), jnp.int32),
    )(xyzp, xyzp)


PB = 256


def _sigmoid(v):
    return 1.0 / (1.0 + jnp.exp(-v))


def _spike_fn(v):
    vc = jnp.clip(v, -10.0, 10.0)
    return 0.5 * jnp.exp(-vc * vc * 0.5) * INV_SQRT_2PI + 0.5 * _sigmoid(10.0 * vc)


def _lif_rows(x, mdv, tav, rdv, tb):
    mem = jnp.zeros_like(x)
    thr = jnp.broadcast_to(tb, x.shape)
    refr = jnp.zeros_like(x)
    acc = jnp.zeros_like(x)
    for _ in range(T):
        xi = x * (refr <= 0.0).astype(x.dtype)
        mem = mem * mdv * (1.0 - refr) + xi
        sp = _spike_fn(mem - thr)
        mem = mem * (1.0 - sp)
        refr = refr * rdv + sp
        thr = thr + tav * sp
        thr = tb + (thr - tb) * 0.95
        acc = acc + sp
    return acc * (1.0 / T)


def _feat_body(x_ref, wfc1_ref, alpha_ref, beta_ref, md_ref, ta_ref, rd_ref,
               tb_ref, wq_ref, wk_ref, wv_ref, q_ref, kv_ref):
    xb = x_ref[0]
    pre = lax.dot_general(xb, wfc1_ref[...], (((0,), (1,)), ((), ())),
                          preferred_element_type=jnp.float32)
    pre = pre * alpha_ref[...] + beta_ref[...]
    mdv = jnp.clip(md_ref[...], 0.1, 0.99)
    tav = jnp.clip(ta_ref[...], 0.001, 0.1)
    rdv = jnp.clip(rd_ref[...], 0.1, 0.95)
    f = _lif_rows(pre, mdv, tav, rdv, tb_ref[...])
    nt = (((1,), (1,)), ((), ()))
    q_ref[0] = lax.dot_general(f, wq_ref[...], nt,
                               preferred_element_type=jnp.float32)
    kf = lax.dot_general(f, wk_ref[...], nt,
                         preferred_element_type=jnp.float32)
    vf = lax.dot_general(f, wv_ref[...], nt,
                         preferred_element_type=jnp.float32)
    k16 = lax.bitcast_convert_type(kf.astype(jnp.bfloat16), jnp.uint16)
    v16 = lax.bitcast_convert_type(vf.astype(jnp.bfloat16), jnp.uint16)
    packed = (v16.astype(jnp.uint32) << 16) | k16.astype(jnp.uint32)
    kv_ref[0] = lax.bitcast_convert_type(packed, jnp.int32)


def _features(x, w_fc1, alpha1, beta1, md, ta, rd, tb, w_q, w_k, w_v):
    full = lambda a: pl.BlockSpec(a.shape, lambda b, n: (0,) * a.ndim)
    return pl.pallas_call(
        _feat_body,
        grid=(B, N // PB),
        in_specs=[
            pl.BlockSpec((1, DP, PB), lambda b, n: (b, 0, n)),
            full(w_fc1), full(alpha1), full(beta1), full(md), full(ta),
            full(rd), full(tb), full(w_q), full(w_k), full(w_v),
        ],
        out_specs=[
            pl.BlockSpec((1, PB, DM), lambda b, n: (b, n, 0)),
            pl.BlockSpec((1, PB, DM), lambda b, n: (b, n, 0)),
        ],
        out_shape=[
            jax.ShapeDtypeStruct((B, N, DM), jnp.float32),
            jax.ShapeDtypeStruct((B, N, DM), jnp.int32),
        ],
    )(x, w_fc1, alpha1, beta1, md, ta, rd, tb, w_q, w_k, w_v)


NW = 32
CH = 128


def _sc_gather_kv(kv_flat, idxg):
    rows = idxg.shape[0]
    rpw = rows // NW
    nch = rpw // CH
    mesh = plsc.VectorSubcoreMesh(core_axis_name="c", subcore_axis_name="s")

    @functools.partial(
        pl.kernel, mesh=mesh,
        out_type=jax.ShapeDtypeStruct((rows, DM), jnp.int32),
        scratch_types=[
            pltpu.VMEM((CH,), jnp.int32),
            pltpu.VMEM((CH, DM), jnp.int32),
            pltpu.SemaphoreType.DMA,
        ],
    )
    def gather(kv_hbm, idx_hbm, kvn_hbm, idx_v, kv_v, sem1):
        wid = lax.axis_index("s") * 2 + lax.axis_index("c")
        base = wid * rpw

        def body(i, carry):
            off = base + i * CH
            pltpu.sync_copy(idx_hbm.at[pl.ds(off, CH)], idx_v)
            pltpu.async_copy(kv_hbm.at[idx_v], kv_v, sem1).wait()
            pltpu.sync_copy(kv_v, kvn_hbm.at[pl.ds(off, CH)])
            return carry

        lax.fori_loop(0, nch, body, 0)

    return gather(kv_flat, idxg)


def _sc_gather_xyz(xyzp_vec, idxg):
    rows = idxg.shape[0]
    rpw = rows // NW
    nch = rpw // CH
    mesh = plsc.VectorSubcoreMesh(core_axis_name="c", subcore_axis_name="s")

    @functools.partial(
        pl.kernel, mesh=mesh,
        out_type=jax.ShapeDtypeStruct((rows * 8,), jnp.float32),
        scratch_types=[
            pltpu.VMEM((B * N * 8,), jnp.float32),
            pltpu.VMEM((CH,), jnp.int32),
            pltpu.VMEM((CH * 8,), jnp.float32),
        ],
        compiler_params=pltpu.CompilerParams(needs_layout_passes=False),
    )
    def gather(xyzp_hbm, idx_hbm, xyzn_hbm, tab_v, idx_v, out_v):
        wid = lax.axis_index("s") * 2 + lax.axis_index("c")
        base = wid * rpw
        pltpu.sync_copy(xyzp_hbm, tab_v)
        lane8 = lax.iota(jnp.int32, 16) * 8

        def body(i, carry):
            off = base + i * CH
            pltpu.sync_copy(idx_hbm.at[pl.ds(off, CH)], idx_v)
            for jj in range(CH // 16):
                rbase = idx_v[pl.ds(jj * 16, 16)] * 8
                for c in range(8):
                    vals = plsc.load_gather(tab_v, [rbase + c])
                    plsc.store_scatter(out_v, [lane8 + (jj * 128 + c)], vals)
            pltpu.sync_copy(out_v, xyzn_hbm.at[pl.ds(off * 8, CH * 8)])
            return carry

        lax.fori_loop(0, nch, body, 0)

    return gather(xyzp_vec, idxg)


def _sc_gather(kv_flat, xyzp_flat, idxg):
    kvn = _sc_gather_kv(kv_flat, idxg)
    xyzn = _sc_gather_xyz(xyzp_flat.reshape(-1), idxg)
    return kvn, xyzn.reshape(-1, 8)


PD = 128
S = PD * K


def _attn_body(q_ref, kvn_ref, xyzn_ref, xq_ref, x_ref, gcat_ref, bcat_ref,
               wd1c_ref, bd1t_ref, wd2_ref, bd2_ref, wg1_ref, bg1_ref,
               wg2_ref, bg2_ref, wfc2_ref, beta2_ref, out_ref):
    kv4 = lax.bitcast_convert_type(kvn_ref[0], jnp.uint32)
    xyzn4 = xyzn_ref[0]
    xq = xq_ref[0]
    q = q_ref[0]
    nt = (((1,), (1,)), ((), ()))
    nn = (((1,), (0,)), ((), ()))

    def unpack_k(u):
        return lax.bitcast_convert_type(
            u.astype(jnp.uint16), jnp.bfloat16).astype(jnp.float32)

    def unpack_v(u):
        return lax.bitcast_convert_type(
            (u >> 16).astype(jnp.uint16), jnp.bfloat16).astype(jnp.float32)

    rel_cat = jnp.concatenate([xq - xyzn4[k] for k in range(K)], axis=1)
    h1_cat = lax.dot_general(rel_cat, wd1c_ref[...], nn,
                             preferred_element_type=jnp.float32)
    h1_cat = jnp.maximum(h1_cat + bd1t_ref[...], 0.0)

    prod_cat = jnp.concatenate([q * unpack_k(kv4[k]) for k in range(K)],
                               axis=1)
    logits = lax.dot_general(prod_cat, gcat_ref[...], nn,
                             preferred_element_type=jnp.float32)
    l3 = (logits[:, :K * H] * (1.0 / (HD ** 0.5))).reshape(PD, K, H)
    m = jnp.max(l3, axis=1, keepdims=True)
    e = jnp.exp(l3 - m)
    ssum = jnp.sum(e, axis=1, keepdims=True)
    attn = (e / ssum).reshape(PD, K * H)
    wf_cat = lax.dot_general(attn, bcat_ref[...], nn,
                             preferred_element_type=jnp.float32)

    vsum = jnp.zeros((PD, DM), jnp.float32)
    asum = jnp.zeros((PD, DM), jnp.float32)
    for k in range(K):
        wf = wf_cat[:, k * DM:(k + 1) * DM]
        vsum = vsum + wf * unpack_v(kv4[k])
        asum = asum + wf * h1_cat[:, k * DM:(k + 1) * DM]
    attn_out = vsum + lax.dot_general(asum, wd2_ref[...], nt,
                                      preferred_element_type=jnp.float32)
    attn_out = attn_out + bd2_ref[...]

    g = lax.dot_general(attn_out, wg1_ref[...], nt,
                        preferred_element_type=jnp.float32)
    g = jnp.maximum(g + bg1_ref[...], 0.0)
    g = lax.dot_general(g, wg2_ref[...], nt,
                        preferred_element_type=jnp.float32) + bg2_ref[...]
    res = lax.dot_general(wfc2_ref[...], g, nt,
                          preferred_element_type=jnp.float32)
    out_ref[0] = res + beta2_ref[...] + x_ref[0]


def _attention(q, kvn, xyzn, xyzp, x, gcat, bcat, wd1c, bd1t, w_d2, b_d2,
               w_g1, b_g1, w_g2, b_g2, wfc2s, beta2):
    full = lambda a: pl.BlockSpec(a.shape, lambda b, n: (0,) * a.ndim)
    return pl.pallas_call(
        _attn_body,
        grid=(B, N // PD),
        in_specs=[
            pl.BlockSpec((1, PD, DM), lambda b, n: (b, n, 0)),
            pl.BlockSpec((1, K, PD, DM), lambda b, n: (b, 0, n, 0)),
            pl.BlockSpec((1, K, PD, 8), lambda b, n: (b, 0, n, 0)),
            pl.BlockSpec((1, PD, 8), lambda b, n: (b, n, 0)),
            pl.BlockSpec((1, DP, PD), lambda b, n: (b, 0, n)),
            full(gcat), full(bcat), full(wd1c), full(bd1t), full(w_d2),
            full(b_d2), full(w_g1), full(b_g1), full(w_g2), full(b_g2),
            full(wfc2s), full(beta2),
        ],
        out_specs=pl.BlockSpec((1, DP, PD), lambda b, n: (b, 0, n)),
        out_shape=jax.ShapeDtypeStruct((B, DP, N), jnp.float32),
    )(q, kvn, xyzn, xyzp, x, gcat, bcat, wd1c, bd1t, w_d2, b_d2, w_g1, b_g1,
      w_g2, b_g2, wfc2s, beta2)


def kernel(x, xyz, w_fc1, b_fc1, bn1_g, bn1_b, mem_decay, thr_adapt,
           refr_decay, thr_base, w_q, w_k, w_v, w_d1, b_d1, w_d2, b_d2,
           w_g1, b_g1, w_g2, b_g2, w_fc2, b_fc2, bn2_g, bn2_b):
    xyzp = jnp.concatenate(
        [xyz, jnp.zeros((B, N, 5), jnp.float32)], axis=-1)

    idx = _knn_topk(xyzp)

    alpha1 = (bn1_g * BN_SCALE).reshape(1, DM)
    beta1 = (b_fc1 * bn1_g * BN_SCALE + bn1_b).reshape(1, DM)
    q, kv = _features(x, w_fc1, alpha1, beta1,
                      mem_decay.reshape(1, DM), thr_adapt.reshape(1, DM),
                      refr_decay.reshape(1, DM), thr_base.reshape(1, DM),
                      w_q, w_k, w_v)

    idxg = idx.transpose(0, 2, 1).reshape(B * N * K)
    kvn_flat, xyzn_flat = _sc_gather(
        kv.reshape(B * N, DM), xyzp.reshape(B * N, 8), idxg)

    wd1p = jnp.concatenate([w_d1, jnp.zeros((DM, 5), jnp.float32)], axis=-1)
    wd1c = jnp.kron(jnp.eye(K, dtype=jnp.float32), wd1p.T)
    bd1t = jnp.tile(b_d1.reshape(1, DM), (1, K))
    h_of_m = jnp.arange(DM, dtype=jnp.int32) // HD
    g3 = jax.nn.one_hot(H * jnp.arange(K, dtype=jnp.int32)[:, None]
                        + h_of_m[None, :], 128, dtype=jnp.float32)
    gcat = g3.reshape(K * DM, 128)
    bcat = gcat[:, :K * H].T

    alpha2 = bn2_g * BN_SCALE
    wfc2s = w_fc2 * alpha2[:, None]
    beta2 = (b_fc2 * alpha2 + bn2_b).reshape(DP, 1)

    return _attention(q, kvn_flat.reshape(B, K, N, DM),
                      xyzn_flat.reshape(B, K, N, 8), xyzp, x,
                      gcat, bcat, wd1c, bd1t, w_d2, b_d2.reshape(1, DM),
                      w_g1, b_g1.reshape(1, DM), w_g2, b_g2.reshape(1, DM),
                      wfc2s, beta2)


# speedup vs baseline: 15.7172x; 1.0515x over previous
import functools

import jax
import jax.numpy as jnp
from jax import lax
from jax.experimental import pallas as pl
from jax.experimental.pallas import tpu as pltpu
from jax.experimental.pallas import tpu_sc as plsc

B, N, DP, DM, K, H, T = 4, 2048, 128, 256, 16, 4, 4
HD = DM // H
INV_SQRT_2PI = 0.3989422804014327
BN_SCALE = 1.0 / (1.0 + 1e-5) ** 0.5

PA = 256


def _knn_body(xt_ref, xf_ref, idx_ref):
    xt = xt_ref[0]
    xf = xf_ref[0]
    s = lax.dot_general(2.0 * xt, xf, (((1,), (1,)), ((), ())),
                        preferred_element_type=jnp.float32)
    ones = jnp.ones((1, 8), jnp.float32)
    xx = lax.dot_general(ones, xf * xf, (((1,), (1,)), ((), ())),
                         preferred_element_type=jnp.float32)
    s = s - xx
    iota = lax.broadcasted_iota(jnp.int32, (PA, N), 1)
    cols = []
    for _ in range(K):
        m = jnp.max(s, axis=1, keepdims=True)
        msk = s >= m
        cand = jnp.where(msk, iota, N)
        a = jnp.min(cand, axis=1, keepdims=True)
        s = jnp.where(msk, -3.0e38, s)
        cols.append(a)
    idx_ref[0] = jnp.concatenate(cols, axis=1)


def _knn_topk(xyzp):
    return pl.pallas_call(
        _knn_body,
        grid=(1, N // PA),
        in_specs=[
            pl.BlockSpec((1, PA, 8), lambda b, n: (b, n, 0)),
            pl.BlockSpec((1, N, 8), lambda b, n: (b, 0, 0)),
        ],
        out_specs=pl.BlockSpec((1, PA, K), lambda b, n: (b, n, 0)),
        out_shape=jax.ShapeDtypeStruct((1, N, K), jnp.int32),
    )(xyzp, xyzp)


PB = 256


def _sigmoid(v):
    return 1.0 / (1.0 + jnp.exp(-v))


def _spike_fn(v):
    vc = jnp.clip(v, -10.0, 10.0)
    return 0.5 * jnp.exp(-vc * vc * 0.5) * INV_SQRT_2PI + 0.5 * _sigmoid(10.0 * vc)


def _lif_rows(x, mdv, tav, rdv, tb):
    mem = jnp.zeros_like(x)
    thr = jnp.broadcast_to(tb, x.shape)
    refr = jnp.zeros_like(x)
    acc = jnp.zeros_like(x)
    for _ in range(T):
        xi = x * (refr <= 0.0).astype(x.dtype)
        mem = mem * mdv * (1.0 - refr) + xi
        sp = _spike_fn(mem - thr)
        mem = mem * (1.0 - sp)
        refr = refr * rdv + sp
        thr = thr + tav * sp
        thr = tb + (thr - tb) * 0.95
        acc = acc + sp
    return acc * (1.0 / T)


def _feat_body(x_ref, wfc1_ref, alpha_ref, beta_ref, md_ref, ta_ref, rd_ref,
               tb_ref, wq_ref, wk_ref, wv_ref, q_ref, kv_ref):
    xb = x_ref[0]
    pre = lax.dot_general(xb, wfc1_ref[...], (((0,), (1,)), ((), ())),
                          preferred_element_type=jnp.float32)
    pre = pre * alpha_ref[...] + beta_ref[...]
    mdv = jnp.clip(md_ref[...], 0.1, 0.99)
    tav = jnp.clip(ta_ref[...], 0.001, 0.1)
    rdv = jnp.clip(rd_ref[...], 0.1, 0.95)
    f = _lif_rows(pre, mdv, tav, rdv, tb_ref[...])
    nt = (((1,), (1,)), ((), ()))
    q_ref[0] = lax.dot_general(f, wq_ref[...], nt,
                               preferred_element_type=jnp.float32)
    kf = lax.dot_general(f, wk_ref[...], nt,
                         preferred_element_type=jnp.float32)
    vf = lax.dot_general(f, wv_ref[...], nt,
                         preferred_element_type=jnp.float32)
    k16 = lax.bitcast_convert_type(kf.astype(jnp.bfloat16), jnp.uint16)
    v16 = lax.bitcast_convert_type(vf.astype(jnp.bfloat16), jnp.uint16)
    packed = (v16.astype(jnp.uint32) << 16) | k16.astype(jnp.uint32)
    kv_ref[0] = lax.bitcast_convert_type(packed, jnp.int32)


def _features(x, w_fc1, alpha1, beta1, md, ta, rd, tb, w_q, w_k, w_v):
    full = lambda a: pl.BlockSpec(a.shape, lambda b, n: (0,) * a.ndim)
    return pl.pallas_call(
        _feat_body,
        grid=(1, N // PB),
        in_specs=[
            pl.BlockSpec((1, DP, PB), lambda b, n: (b, 0, n)),
            full(w_fc1), full(alpha1), full(beta1), full(md), full(ta),
            full(rd), full(tb), full(w_q), full(w_k), full(w_v),
        ],
        out_specs=[
            pl.BlockSpec((1, PB, DM), lambda b, n: (b, n, 0)),
            pl.BlockSpec((1, PB, DM), lambda b, n: (b, n, 0)),
        ],
        out_shape=[
            jax.ShapeDtypeStruct((1, N, DM), jnp.float32),
            jax.ShapeDtypeStruct((1, N, DM), jnp.int32),
        ],
    )(x, w_fc1, alpha1, beta1, md, ta, rd, tb, w_q, w_k, w_v)


NW = 32
CH = 128


def _sc_gather_kv(kv_flat, idxg):
    rows = idxg.shape[0]
    rpw = rows // NW
    nch = rpw // CH
    mesh = plsc.VectorSubcoreMesh(core_axis_name="c", subcore_axis_name="s")

    @functools.partial(
        pl.kernel, mesh=mesh,
        out_type=jax.ShapeDtypeStruct((rows, DM), jnp.int32),
        scratch_types=[
            pltpu.VMEM((CH,), jnp.int32),
            pltpu.VMEM((CH, DM), jnp.int32),
            pltpu.SemaphoreType.DMA,
        ],
    )
    def gather(kv_hbm, idx_hbm, kvn_hbm, idx_v, kv_v, sem1):
        wid = lax.axis_index("s") * 2 + lax.axis_index("c")
        base = wid * rpw

        def body(i, carry):
            off = base + i * CH
            pltpu.sync_copy(idx_hbm.at[pl.ds(off, CH)], idx_v)
            pltpu.async_copy(kv_hbm.at[idx_v], kv_v, sem1).wait()
            pltpu.sync_copy(kv_v, kvn_hbm.at[pl.ds(off, CH)])
            return carry

        lax.fori_loop(0, nch, body, 0)

    return gather(kv_flat, idxg)


def _sc_gather_xyz(xyzp_vec, idxg):
    rows = idxg.shape[0]
    rpw = rows // NW
    nch = rpw // CH
    mesh = plsc.VectorSubcoreMesh(core_axis_name="c", subcore_axis_name="s")

    @functools.partial(
        pl.kernel, mesh=mesh,
        out_type=jax.ShapeDtypeStruct((rows * 8,), jnp.float32),
        scratch_types=[
            pltpu.VMEM((N * 8,), jnp.float32),
            pltpu.VMEM((CH,), jnp.int32),
            pltpu.VMEM((CH * 8,), jnp.float32),
        ],
        compiler_params=pltpu.CompilerParams(needs_layout_passes=False),
    )
    def gather(xyzp_hbm, idx_hbm, xyzn_hbm, tab_v, idx_v, out_v):
        wid = lax.axis_index("s") * 2 + lax.axis_index("c")
        base = wid * rpw
        pltpu.sync_copy(xyzp_hbm, tab_v)
        lane8 = lax.iota(jnp.int32, 16) * 8

        def body(i, carry):
            off = base + i * CH
            pltpu.sync_copy(idx_hbm.at[pl.ds(off, CH)], idx_v)
            for jj in range(CH // 16):
                rbase = idx_v[pl.ds(jj * 16, 16)] * 8
                for c in range(8):
                    vals = plsc.load_gather(tab_v, [rbase + c])
                    plsc.store_scatter(out_v, [lane8 + (jj * 128 + c)], vals)
            pltpu.sync_copy(out_v, xyzn_hbm.at[pl.ds(off * 8, CH * 8)])
            return carry

        lax.fori_loop(0, nch, body, 0)

    return gather(xyzp_vec, idxg)


def _sc_gather(kv_flat, xyzp_flat, idxg):
    kvn = _sc_gather_kv(kv_flat, idxg)
    xyzn = _sc_gather_xyz(xyzp_flat.reshape(-1), idxg)
    return kvn, xyzn.reshape(-1, 8)


PD = 128
S = PD * K


def _attn_body(q_ref, kvn_ref, xyzn_ref, xq_ref, x_ref, gcat_ref, bcat_ref,
               wd1c_ref, bd1t_ref, wd2_ref, bd2_ref, wg1_ref, bg1_ref,
               wg2_ref, bg2_ref, wfc2_ref, beta2_ref, out_ref):
    kv4 = lax.bitcast_convert_type(kvn_ref[0], jnp.uint32)
    xyzn4 = xyzn_ref[0]
    xq = xq_ref[0]
    q = q_ref[0]
    nt = (((1,), (1,)), ((), ()))
    nn = (((1,), (0,)), ((), ()))

    def unpack_k(u):
        return lax.bitcast_convert_type(
            u.astype(jnp.uint16), jnp.bfloat16).astype(jnp.float32)

    def unpack_v(u):
        return lax.bitcast_convert_type(
            (u >> 16).astype(jnp.uint16), jnp.bfloat16).astype(jnp.float32)

    rel_cat = jnp.concatenate([xq - xyzn4[k] for k in range(K)], axis=1)
    h1_cat = lax.dot_general(rel_cat, wd1c_ref[...], nn,
                             preferred_element_type=jnp.float32)
    h1_cat = jnp.maximum(h1_cat + bd1t_ref[...], 0.0)

    prod_cat = jnp.concatenate([q * unpack_k(kv4[k]) for k in range(K)],
                               axis=1)
    logits = lax.dot_general(prod_cat, gcat_ref[...], nn,
                             preferred_element_type=jnp.float32)
    l3 = (logits[:, :K * H] * (1.0 / (HD ** 0.5))).reshape(PD, K, H)
    m = jnp.max(l3, axis=1, keepdims=True)
    e = jnp.exp(l3 - m)
    ssum = jnp.sum(e, axis=1, keepdims=True)
    attn = (e / ssum).reshape(PD, K * H)
    wf_cat = lax.dot_general(attn, bcat_ref[...], nn,
                             preferred_element_type=jnp.float32)

    vsum = jnp.zeros((PD, DM), jnp.float32)
    asum = jnp.zeros((PD, DM), jnp.float32)
    for k in range(K):
        wf = wf_cat[:, k * DM:(k + 1) * DM]
        vsum = vsum + wf * unpack_v(kv4[k])
        asum = asum + wf * h1_cat[:, k * DM:(k + 1) * DM]
    attn_out = vsum + lax.dot_general(asum, wd2_ref[...], nt,
                                      preferred_element_type=jnp.float32)
    attn_out = attn_out + bd2_ref[...]

    g = lax.dot_general(attn_out, wg1_ref[...], nt,
                        preferred_element_type=jnp.float32)
    g = jnp.maximum(g + bg1_ref[...], 0.0)
    g = lax.dot_general(g, wg2_ref[...], nt,
                        preferred_element_type=jnp.float32) + bg2_ref[...]
    res = lax.dot_general(wfc2_ref[...], g, nt,
                          preferred_element_type=jnp.float32)
    out_ref[0] = res + beta2_ref[...] + x_ref[0]


def _attention(q, kvn, xyzn, xyzp, x, gcat, bcat, wd1c, bd1t, w_d2, b_d2,
               w_g1, b_g1, w_g2, b_g2, wfc2s, beta2):
    full = lambda a: pl.BlockSpec(a.shape, lambda b, n: (0,) * a.ndim)
    return pl.pallas_call(
        _attn_body,
        grid=(1, N // PD),
        in_specs=[
            pl.BlockSpec((1, PD, DM), lambda b, n: (b, n, 0)),
            pl.BlockSpec((1, K, PD, DM), lambda b, n: (b, 0, n, 0)),
            pl.BlockSpec((1, K, PD, 8), lambda b, n: (b, 0, n, 0)),
            pl.BlockSpec((1, PD, 8), lambda b, n: (b, n, 0)),
            pl.BlockSpec((1, DP, PD), lambda b, n: (b, 0, n)),
            full(gcat), full(bcat), full(wd1c), full(bd1t), full(w_d2),
            full(b_d2), full(w_g1), full(b_g1), full(w_g2), full(b_g2),
            full(wfc2s), full(beta2),
        ],
        out_specs=pl.BlockSpec((1, DP, PD), lambda b, n: (b, 0, n)),
        out_shape=jax.ShapeDtypeStruct((1, DP, N), jnp.float32),
    )(q, kvn, xyzn, xyzp, x, gcat, bcat, wd1c, bd1t, w_d2, b_d2, w_g1, b_g1,
      w_g2, b_g2, wfc2s, beta2)


def kernel(x, xyz, w_fc1, b_fc1, bn1_g, bn1_b, mem_decay, thr_adapt,
           refr_decay, thr_base, w_q, w_k, w_v, w_d1, b_d1, w_d2, b_d2,
           w_g1, b_g1, w_g2, b_g2, w_fc2, b_fc2, bn2_g, bn2_b):
    xyzp = jnp.concatenate(
        [xyz, jnp.zeros((B, N, 5), jnp.float32)], axis=-1)

    alpha1 = (bn1_g * BN_SCALE).reshape(1, DM)
    beta1 = (b_fc1 * bn1_g * BN_SCALE + bn1_b).reshape(1, DM)
    mdp = mem_decay.reshape(1, DM)
    tap = thr_adapt.reshape(1, DM)
    rdp = refr_decay.reshape(1, DM)
    tbp = thr_base.reshape(1, DM)

    wd1p = jnp.concatenate([w_d1, jnp.zeros((DM, 5), jnp.float32)], axis=-1)
    wd1c = jnp.kron(jnp.eye(K, dtype=jnp.float32), wd1p.T)
    bd1t = jnp.tile(b_d1.reshape(1, DM), (1, K))
    h_of_m = jnp.arange(DM, dtype=jnp.int32) // HD
    g3 = jax.nn.one_hot(H * jnp.arange(K, dtype=jnp.int32)[:, None]
                        + h_of_m[None, :], 128, dtype=jnp.float32)
    gcat = g3.reshape(K * DM, 128)
    bcat = gcat[:, :K * H].T

    alpha2 = bn2_g * BN_SCALE
    wfc2s = w_fc2 * alpha2[:, None]
    beta2 = (b_fc2 * alpha2 + bn2_b).reshape(DP, 1)

    outs = []
    for b in range(B):
        xyzp_b = lax.slice_in_dim(xyzp, b, b + 1, axis=0)
        x_b = lax.slice_in_dim(x, b, b + 1, axis=0)
        idx_b = _knn_topk(xyzp_b)
        q_b, kv_b = _features(x_b, w_fc1, alpha1, beta1, mdp, tap, rdp,
                              tbp, w_q, w_k, w_v)
        idxg_b = idx_b.reshape(N, K).transpose(1, 0).reshape(N * K)
        kvn_b, xyzn_b = _sc_gather(
            kv_b.reshape(N, DM), xyzp_b.reshape(N, 8), idxg_b)
        outs.append(_attention(
            q_b, kvn_b.reshape(1, K, N, DM), xyzn_b.reshape(1, K, N, 8),
            xyzp_b, x_b, gcat, bcat, wd1c, bd1t, w_d2, b_d2.reshape(1, DM),
            w_g1, b_g1.reshape(1, DM), w_g2, b_g2.reshape(1, DM),
            wfc2s, beta2))
    return jnp.concatenate(outs, axis=0)


# speedup vs baseline: 15.7325x; 1.0010x over previous
import functools

import jax
import jax.numpy as jnp
from jax import lax
from jax.experimental import pallas as pl
from jax.experimental.pallas import tpu as pltpu
from jax.experimental.pallas import tpu_sc as plsc

B, N, DP, DM, K, H, T = 4, 2048, 128, 256, 16, 4, 4
HD = DM // H
INV_SQRT_2PI = 0.3989422804014327
BN_SCALE = 1.0 / (1.0 + 1e-5) ** 0.5

PA = 256


def _knn_body(xt_ref, xf_ref, idx_ref):
    xt = xt_ref[0]
    xf = xf_ref[0]
    s = lax.dot_general(2.0 * xt, xf, (((1,), (1,)), ((), ())),
                        preferred_element_type=jnp.float32)
    ones = jnp.ones((1, 8), jnp.float32)
    xx = lax.dot_general(ones, xf * xf, (((1,), (1,)), ((), ())),
                         preferred_element_type=jnp.float32)
    s = s - xx
    iota = lax.broadcasted_iota(jnp.int32, (PA, N), 1)
    cols = []
    for _ in range(K):
        m = jnp.max(s, axis=1, keepdims=True)
        msk = s >= m
        cand = jnp.where(msk, iota, N)
        a = jnp.min(cand, axis=1, keepdims=True)
        s = jnp.where(msk, -3.0e38, s)
        cols.append(a)
    idx_ref[0] = jnp.concatenate(cols, axis=1)


def _knn_topk(xyzp):
    return pl.pallas_call(
        _knn_body,
        grid=(1, N // PA),
        in_specs=[
            pl.BlockSpec((1, PA, 8), lambda b, n: (b, n, 0)),
            pl.BlockSpec((1, N, 8), lambda b, n: (b, 0, 0)),
        ],
        out_specs=pl.BlockSpec((1, PA, K), lambda b, n: (b, n, 0)),
        out_shape=jax.ShapeDtypeStruct((1, N, K), jnp.int32),
    )(xyzp, xyzp)


PB = 256


def _sigmoid(v):
    return 1.0 / (1.0 + jnp.exp(-v))


def _spike_fn(v):
    vc = jnp.clip(v, -10.0, 10.0)
    return 0.5 * jnp.exp(-vc * vc * 0.5) * INV_SQRT_2PI + 0.5 * _sigmoid(10.0 * vc)


def _lif_rows(x, mdv, tav, rdv, tb):
    mem = jnp.zeros_like(x)
    thr = jnp.broadcast_to(tb, x.shape)
    refr = jnp.zeros_like(x)
    acc = jnp.zeros_like(x)
    for _ in range(T):
        xi = x * (refr <= 0.0).astype(x.dtype)
        mem = mem * mdv * (1.0 - refr) + xi
        sp = _spike_fn(mem - thr)
        mem = mem * (1.0 - sp)
        refr = refr * rdv + sp
        thr = thr + tav * sp
        thr = tb + (thr - tb) * 0.95
        acc = acc + sp
    return acc * (1.0 / T)


def _feat_body(x_ref, wfc1_ref, alpha_ref, beta_ref, md_ref, ta_ref, rd_ref,
               tb_ref, wq_ref, wk_ref, wv_ref, q_ref, kv_ref):
    xb = x_ref[0]
    pre = lax.dot_general(xb, wfc1_ref[...], (((0,), (1,)), ((), ())),
                          preferred_element_type=jnp.float32)
    pre = pre * alpha_ref[...] + beta_ref[...]
    mdv = jnp.clip(md_ref[...], 0.1, 0.99)
    tav = jnp.clip(ta_ref[...], 0.001, 0.1)
    rdv = jnp.clip(rd_ref[...], 0.1, 0.95)
    f = _lif_rows(pre, mdv, tav, rdv, tb_ref[...])
    nt = (((1,), (1,)), ((), ()))
    q_ref[0] = lax.dot_general(f, wq_ref[...], nt,
                               preferred_element_type=jnp.float32)
    kf = lax.dot_general(f, wk_ref[...], nt,
                         preferred_element_type=jnp.float32)
    vf = lax.dot_general(f, wv_ref[...], nt,
                         preferred_element_type=jnp.float32)
    k16 = lax.bitcast_convert_type(kf.astype(jnp.bfloat16), jnp.uint16)
    v16 = lax.bitcast_convert_type(vf.astype(jnp.bfloat16), jnp.uint16)
    packed = (v16.astype(jnp.uint32) << 16) | k16.astype(jnp.uint32)
    kv_ref[0] = lax.bitcast_convert_type(packed, jnp.int32)


def _features(x, w_fc1, alpha1, beta1, md, ta, rd, tb, w_q, w_k, w_v):
    full = lambda a: pl.BlockSpec(a.shape, lambda b, n: (0,) * a.ndim)
    return pl.pallas_call(
        _feat_body,
        grid=(1, N // PB),
        in_specs=[
            pl.BlockSpec((1, DP, PB), lambda b, n: (b, 0, n)),
            full(w_fc1), full(alpha1), full(beta1), full(md), full(ta),
            full(rd), full(tb), full(w_q), full(w_k), full(w_v),
        ],
        out_specs=[
            pl.BlockSpec((1, PB, DM), lambda b, n: (b, n, 0)),
            pl.BlockSpec((1, PB, DM), lambda b, n: (b, n, 0)),
        ],
        out_shape=[
            jax.ShapeDtypeStruct((1, N, DM), jnp.float32),
            jax.ShapeDtypeStruct((1, N, DM), jnp.int32),
        ],
    )(x, w_fc1, alpha1, beta1, md, ta, rd, tb, w_q, w_k, w_v)


NW = 32
CH = 128


def _sc_gather_fused(kv_flat, xyzp_vec, idxg):
    rows = idxg.shape[0]
    rpw = rows // NW
    nch = rpw // CH
    mesh = plsc.VectorSubcoreMesh(core_axis_name="c", subcore_axis_name="s")

    @functools.partial(
        pl.kernel, mesh=mesh,
        out_type=[
            jax.ShapeDtypeStruct((rows, DM), jnp.int32),
            jax.ShapeDtypeStruct((rows * 8,), jnp.float32),
        ],
        scratch_types=[
            pltpu.VMEM((N * 8,), jnp.float32),
            pltpu.VMEM((CH,), jnp.int32),
            pltpu.VMEM((CH,), jnp.int32),
            pltpu.VMEM((CH, DM), jnp.int32),
            pltpu.VMEM((CH, DM), jnp.int32),
            pltpu.VMEM((CH * 8,), jnp.float32),
            pltpu.VMEM((CH * 8,), jnp.float32),
            pltpu.SemaphoreType.DMA,
            pltpu.SemaphoreType.DMA,
        ],
        compiler_params=pltpu.CompilerParams(needs_layout_passes=False),
    )
    def gather(kv_hbm, xyzp_hbm, idx_hbm, kvn_hbm, xyzn_hbm,
               tab_v, idx_v0, idx_v1, kv_v0, kv_v1, out_v0, out_v1,
               sem0, sem1):
        wid = lax.axis_index("s") * 2 + lax.axis_index("c")
        base = wid * rpw
        pltpu.sync_copy(xyzp_hbm, tab_v)
        lane8 = lax.iota(jnp.int32, 16) * 8
        idx_bufs = [idx_v0, idx_v1]
        kv_bufs = [kv_v0, kv_v1]
        out_bufs = [out_v0, out_v1]
        sems = [sem0, sem1]

        def start(c):
            p = c % 2
            off = base + c * CH
            pltpu.sync_copy(idx_hbm.at[pl.ds(off, CH)], idx_bufs[p])
            return pltpu.async_copy(kv_hbm.at[idx_bufs[p]], kv_bufs[p],
                                    sems[p])

        cps = {0: start(0)}
        for c in range(nch):
            p = c % 2
            off = base + c * CH
            if c + 1 < nch:
                cps[c + 1] = start(c + 1)
            out_v = out_bufs[p]
            idx_v = idx_bufs[p]
            for jj in range(CH // 16):
                rbase = idx_v[pl.ds(jj * 16, 16)] * 8
                for cc in range(8):
                    vals = plsc.load_gather(tab_v, [rbase + cc])
                    plsc.store_scatter(out_v, [lane8 + (jj * 128 + cc)],
                                       vals)
            pltpu.sync_copy(out_v, xyzn_hbm.at[pl.ds(off * 8, CH * 8)])
            cps[c].wait()
            pltpu.sync_copy(kv_bufs[p], kvn_hbm.at[pl.ds(off, CH)])

    return gather(kv_flat, xyzp_vec, idxg)


def _sc_gather(kv_flat, xyzp_flat, idxg):
    kvn, xyzn = _sc_gather_fused(kv_flat, xyzp_flat.reshape(-1), idxg)
    return kvn, xyzn.reshape(-1, 8)


PD = 128
S = PD * K


def _attn_body(q_ref, kvn_ref, xyzn_ref, xq_ref, x_ref, gcat_ref, bcat_ref,
               wd1c_ref, bd1t_ref, wd2_ref, bd2_ref, wg1_ref, bg1_ref,
               wg2_ref, bg2_ref, wfc2_ref, beta2_ref, out_ref):
    kv4 = lax.bitcast_convert_type(kvn_ref[0], jnp.uint32)
    xyzn4 = xyzn_ref[0]
    xq = xq_ref[0]
    q = q_ref[0]
    nt = (((1,), (1,)), ((), ()))
    nn = (((1,), (0,)), ((), ()))

    def unpack_k(u):
        return lax.bitcast_convert_type(
            u.astype(jnp.uint16), jnp.bfloat16).astype(jnp.float32)

    def unpack_v(u):
        return lax.bitcast_convert_type(
            (u >> 16).astype(jnp.uint16), jnp.bfloat16).astype(jnp.float32)

    rel_cat = jnp.concatenate([xq - xyzn4[k] for k in range(K)], axis=1)
    h1_cat = lax.dot_general(rel_cat, wd1c_ref[...], nn,
                             preferred_element_type=jnp.float32)
    h1_cat = jnp.maximum(h1_cat + bd1t_ref[...], 0.0)

    prod_cat = jnp.concatenate([q * unpack_k(kv4[k]) for k in range(K)],
                               axis=1)
    logits = lax.dot_general(prod_cat, gcat_ref[...], nn,
                             preferred_element_type=jnp.float32)
    l3 = (logits[:, :K * H] * (1.0 / (HD ** 0.5))).reshape(PD, K, H)
    m = jnp.max(l3, axis=1, keepdims=True)
    e = jnp.exp(l3 - m)
    ssum = jnp.sum(e, axis=1, keepdims=True)
    attn = (e / ssum).reshape(PD, K * H)
    wf_cat = lax.dot_general(attn, bcat_ref[...], nn,
                             preferred_element_type=jnp.float32)

    vsum = jnp.zeros((PD, DM), jnp.float32)
    asum = jnp.zeros((PD, DM), jnp.float32)
    for k in range(K):
        wf = wf_cat[:, k * DM:(k + 1) * DM]
        vsum = vsum + wf * unpack_v(kv4[k])
        asum = asum + wf * h1_cat[:, k * DM:(k + 1) * DM]
    attn_out = vsum + lax.dot_general(asum, wd2_ref[...], nt,
                                      preferred_element_type=jnp.float32)
    attn_out = attn_out + bd2_ref[...]

    g = lax.dot_general(attn_out, wg1_ref[...], nt,
                        preferred_element_type=jnp.float32)
    g = jnp.maximum(g + bg1_ref[...], 0.0)
    g = lax.dot_general(g, wg2_ref[...], nt,
                        preferred_element_type=jnp.float32) + bg2_ref[...]
    res = lax.dot_general(wfc2_ref[...], g, nt,
                          preferred_element_type=jnp.float32)
    out_ref[0] = res + beta2_ref[...] + x_ref[0]


def _attention(q, kvn, xyzn, xyzp, x, gcat, bcat, wd1c, bd1t, w_d2, b_d2,
               w_g1, b_g1, w_g2, b_g2, wfc2s, beta2):
    full = lambda a: pl.BlockSpec(a.shape, lambda b, n: (0,) * a.ndim)
    return pl.pallas_call(
        _attn_body,
        grid=(1, N // PD),
        in_specs=[
            pl.BlockSpec((1, PD, DM), lambda b, n: (b, n, 0)),
            pl.BlockSpec((1, K, PD, DM), lambda b, n: (b, 0, n, 0)),
            pl.BlockSpec((1, K, PD, 8), lambda b, n: (b, 0, n, 0)),
            pl.BlockSpec((1, PD, 8), lambda b, n: (b, n, 0)),
            pl.BlockSpec((1, DP, PD), lambda b, n: (b, 0, n)),
            full(gcat), full(bcat), full(wd1c), full(bd1t), full(w_d2),
            full(b_d2), full(w_g1), full(b_g1), full(w_g2), full(b_g2),
            full(wfc2s), full(beta2),
        ],
        out_specs=pl.BlockSpec((1, DP, PD), lambda b, n: (b, 0, n)),
        out_shape=jax.ShapeDtypeStruct((1, DP, N), jnp.float32),
    )(q, kvn, xyzn, xyzp, x, gcat, bcat, wd1c, bd1t, w_d2, b_d2, w_g1, b_g1,
      w_g2, b_g2, wfc2s, beta2)


def kernel(x, xyz, w_fc1, b_fc1, bn1_g, bn1_b, mem_decay, thr_adapt,
           refr_decay, thr_base, w_q, w_k, w_v, w_d1, b_d1, w_d2, b_d2,
           w_g1, b_g1, w_g2, b_g2, w_fc2, b_fc2, bn2_g, bn2_b):
    xyzp = jnp.concatenate(
        [xyz, jnp.zeros((B, N, 5), jnp.float32)], axis=-1)

    alpha1 = (bn1_g * BN_SCALE).reshape(1, DM)
    beta1 = (b_fc1 * bn1_g * BN_SCALE + bn1_b).reshape(1, DM)
    mdp = mem_decay.reshape(1, DM)
    tap = thr_adapt.reshape(1, DM)
    rdp = refr_decay.reshape(1, DM)
    tbp = thr_base.reshape(1, DM)

    wd1p = jnp.concatenate([w_d1, jnp.zeros((DM, 5), jnp.float32)], axis=-1)
    wd1c = jnp.kron(jnp.eye(K, dtype=jnp.float32), wd1p.T)
    bd1t = jnp.tile(b_d1.reshape(1, DM), (1, K))
    h_of_m = jnp.arange(DM, dtype=jnp.int32) // HD
    g3 = jax.nn.one_hot(H * jnp.arange(K, dtype=jnp.int32)[:, None]
                        + h_of_m[None, :], 128, dtype=jnp.float32)
    gcat = g3.reshape(K * DM, 128)
    bcat = gcat[:, :K * H].T

    alpha2 = bn2_g * BN_SCALE
    wfc2s = w_fc2 * alpha2[:, None]
    beta2 = (b_fc2 * alpha2 + bn2_b).reshape(DP, 1)

    outs = []
    for b in range(B):
        xyzp_b = lax.slice_in_dim(xyzp, b, b + 1, axis=0)
        x_b = lax.slice_in_dim(x, b, b + 1, axis=0)
        idx_b = _knn_topk(xyzp_b)
        q_b, kv_b = _features(x_b, w_fc1, alpha1, beta1, mdp, tap, rdp,
                              tbp, w_q, w_k, w_v)
        idxg_b = idx_b.reshape(N, K).transpose(1, 0).reshape(N * K)
        kvn_b, xyzn_b = _sc_gather(
            kv_b.reshape(N, DM), xyzp_b.reshape(N, 8), idxg_b)
        outs.append(_attention(
            q_b, kvn_b.reshape(1, K, N, DM), xyzn_b.reshape(1, K, N, 8),
            xyzp_b, x_b, gcat, bcat, wd1c, bd1t, w_d2, b_d2.reshape(1, DM),
            w_g1, b_g1.reshape(1, DM), w_g2, b_g2.reshape(1, DM),
            wfc2s, beta2))
    return jnp.concatenate(outs, axis=0)


# speedup vs baseline: 16.9148x; 1.0751x over previous
import functools

import jax
import jax.numpy as jnp
from jax import lax
from jax.experimental import pallas as pl
from jax.experimental.pallas import tpu as pltpu
from jax.experimental.pallas import tpu_sc as plsc

B, N, DP, DM, K, H, T = 4, 2048, 128, 256, 16, 4, 4
HD = DM // H
INV_SQRT_2PI = 0.3989422804014327
BN_SCALE = 1.0 / (1.0 + 1e-5) ** 0.5

PA = 256


PB = 256


def _sigmoid(v):
    return 1.0 / (1.0 + jnp.exp(-v))


def _spike_fn(v):
    vc = jnp.clip(v, -10.0, 10.0)
    return 0.5 * jnp.exp(-vc * vc * 0.5) * INV_SQRT_2PI + 0.5 * _sigmoid(10.0 * vc)


def _lif_rows(x, mdv, tav, rdv, tb):
    mem = jnp.zeros_like(x)
    thr = jnp.broadcast_to(tb, x.shape)
    refr = jnp.zeros_like(x)
    acc = jnp.zeros_like(x)
    for _ in range(T):
        xi = x * (refr <= 0.0).astype(x.dtype)
        mem = mem * mdv * (1.0 - refr) + xi
        sp = _spike_fn(mem - thr)
        mem = mem * (1.0 - sp)
        refr = refr * rdv + sp
        thr = thr + tav * sp
        thr = tb + (thr - tb) * 0.95
        acc = acc + sp
    return acc * (1.0 / T)


def _feat_body(xt_ref, xf_ref, x_ref, wfc1_ref, alpha_ref, beta_ref,
               md_ref, ta_ref, rd_ref, tb_ref, wq_ref, wk_ref, wv_ref,
               idx_ref, q_ref, kv_ref):
    xt = xt_ref[0]
    xf = xf_ref[0]
    s = lax.dot_general(2.0 * xt, xf, (((1,), (1,)), ((), ())),
                        preferred_element_type=jnp.float32)
    ones = jnp.ones((1, 8), jnp.float32)
    xx = lax.dot_general(ones, xf * xf, (((1,), (1,)), ((), ())),
                         preferred_element_type=jnp.float32)
    s = s - xx
    iota = lax.broadcasted_iota(jnp.int32, (PB, N), 1)
    cols = []
    for _ in range(K):
        m = jnp.max(s, axis=1, keepdims=True)
        msk = s >= m
        cand = jnp.where(msk, iota, N)
        a = jnp.min(cand, axis=1, keepdims=True)
        s = jnp.where(msk, -3.0e38, s)
        cols.append(a)
    idx_ref[0] = jnp.concatenate(cols, axis=1)

    xb = x_ref[0]
    pre = lax.dot_general(xb, wfc1_ref[...], (((0,), (1,)), ((), ())),
                          preferred_element_type=jnp.float32)
    pre = pre * alpha_ref[...] + beta_ref[...]
    mdv = jnp.clip(md_ref[...], 0.1, 0.99)
    tav = jnp.clip(ta_ref[...], 0.001, 0.1)
    rdv = jnp.clip(rd_ref[...], 0.1, 0.95)
    f = _lif_rows(pre, mdv, tav, rdv, tb_ref[...])
    nt = (((1,), (1,)), ((), ()))
    q_ref[0] = lax.dot_general(f, wq_ref[...], nt,
                               preferred_element_type=jnp.float32)
    kf = lax.dot_general(f, wk_ref[...], nt,
                         preferred_element_type=jnp.float32)
    vf = lax.dot_general(f, wv_ref[...], nt,
                         preferred_element_type=jnp.float32)
    k16 = lax.bitcast_convert_type(kf.astype(jnp.bfloat16), jnp.uint16)
    v16 = lax.bitcast_convert_type(vf.astype(jnp.bfloat16), jnp.uint16)
    packed = (v16.astype(jnp.uint32) << 16) | k16.astype(jnp.uint32)
    kv_ref[0] = lax.bitcast_convert_type(packed, jnp.int32)


def _features(xyzp, x, w_fc1, alpha1, beta1, md, ta, rd, tb, w_q, w_k, w_v):
    full = lambda a: pl.BlockSpec(a.shape, lambda b, n: (0,) * a.ndim)
    return pl.pallas_call(
        _feat_body,
        grid=(1, N // PB),
        in_specs=[
            pl.BlockSpec((1, PB, 8), lambda b, n: (b, n, 0)),
            pl.BlockSpec((1, N, 8), lambda b, n: (b, 0, 0)),
            pl.BlockSpec((1, DP, PB), lambda b, n: (b, 0, n)),
            full(w_fc1), full(alpha1), full(beta1), full(md), full(ta),
            full(rd), full(tb), full(w_q), full(w_k), full(w_v),
        ],
        out_specs=[
            pl.BlockSpec((1, PB, K), lambda b, n: (b, n, 0)),
            pl.BlockSpec((1, PB, DM), lambda b, n: (b, n, 0)),
            pl.BlockSpec((1, PB, DM), lambda b, n: (b, n, 0)),
        ],
        out_shape=[
            jax.ShapeDtypeStruct((1, N, K), jnp.int32),
            jax.ShapeDtypeStruct((1, N, DM), jnp.float32),
            jax.ShapeDtypeStruct((1, N, DM), jnp.int32),
        ],
    )(xyzp, xyzp, x, w_fc1, alpha1, beta1, md, ta, rd, tb, w_q, w_k, w_v)


NW = 32
CH = 128


def _sc_gather_fused(kv_flat, xyzp_vec, idxg):
    rows = idxg.shape[0]
    rpw = rows // NW
    nch = rpw // CH
    mesh = plsc.VectorSubcoreMesh(core_axis_name="c", subcore_axis_name="s")

    @functools.partial(
        pl.kernel, mesh=mesh,
        out_type=[
            jax.ShapeDtypeStruct((rows, DM), jnp.int32),
            jax.ShapeDtypeStruct((rows * 8,), jnp.float32),
        ],
        scratch_types=[
            pltpu.VMEM((N * 8,), jnp.float32),
            pltpu.VMEM((CH,), jnp.int32),
            pltpu.VMEM((CH,), jnp.int32),
            pltpu.VMEM((CH, DM), jnp.int32),
            pltpu.VMEM((CH, DM), jnp.int32),
            pltpu.VMEM((CH * 8,), jnp.float32),
            pltpu.VMEM((CH * 8,), jnp.float32),
            pltpu.SemaphoreType.DMA,
            pltpu.SemaphoreType.DMA,
        ],
        compiler_params=pltpu.CompilerParams(needs_layout_passes=False),
    )
    def gather(kv_hbm, xyzp_hbm, idx_hbm, kvn_hbm, xyzn_hbm,
               tab_v, idx_v0, idx_v1, kv_v0, kv_v1, out_v0, out_v1,
               sem0, sem1):
        wid = lax.axis_index("s") * 2 + lax.axis_index("c")
        base = wid * rpw
        pltpu.sync_copy(xyzp_hbm, tab_v)
        lane8 = lax.iota(jnp.int32, 16) * 8
        idx_bufs = [idx_v0, idx_v1]
        kv_bufs = [kv_v0, kv_v1]
        out_bufs = [out_v0, out_v1]
        sems = [sem0, sem1]

        def start(c):
            p = c % 2
            off = base + c * CH
            pltpu.sync_copy(idx_hbm.at[pl.ds(off, CH)], idx_bufs[p])
            return pltpu.async_copy(kv_hbm.at[idx_bufs[p]], kv_bufs[p],
                                    sems[p])

        cps = {0: start(0)}
        for c in range(nch):
            p = c % 2
            off = base + c * CH
            if c + 1 < nch:
                cps[c + 1] = start(c + 1)
            out_v = out_bufs[p]
            idx_v = idx_bufs[p]
            for jj in range(CH // 16):
                rbase = idx_v[pl.ds(jj * 16, 16)] * 8
                for cc in range(8):
                    vals = plsc.load_gather(tab_v, [rbase + cc])
                    plsc.store_scatter(out_v, [lane8 + (jj * 128 + cc)],
                                       vals)
            pltpu.sync_copy(out_v, xyzn_hbm.at[pl.ds(off * 8, CH * 8)])
            cps[c].wait()
            pltpu.sync_copy(kv_bufs[p], kvn_hbm.at[pl.ds(off, CH)])

    return gather(kv_flat, xyzp_vec, idxg)


def _sc_gather(kv_flat, xyzp_flat, idxg):
    kvn, xyzn = _sc_gather_fused(kv_flat, xyzp_flat.reshape(-1), idxg)
    return kvn, xyzn.reshape(-1, 8)


PD = 128
S = PD * K


def _attn_body(q_ref, kvn_ref, xyzn_ref, xq_ref, x_ref, gcat_ref, bcat_ref,
               wd1c_ref, bd1t_ref, wd2_ref, bd2_ref, wg1_ref, bg1_ref,
               wg2_ref, bg2_ref, wfc2_ref, beta2_ref, out_ref):
    kv4 = lax.bitcast_convert_type(kvn_ref[0], jnp.uint32)
    xyzn4 = xyzn_ref[0]
    xq = xq_ref[0]
    q = q_ref[0]
    nt = (((1,), (1,)), ((), ()))
    nn = (((1,), (0,)), ((), ()))

    def unpack_k(u):
        return lax.bitcast_convert_type(
            u.astype(jnp.uint16), jnp.bfloat16).astype(jnp.float32)

    def unpack_v(u):
        return lax.bitcast_convert_type(
            (u >> 16).astype(jnp.uint16), jnp.bfloat16).astype(jnp.float32)

    rel_cat = jnp.concatenate([xq - xyzn4[k] for k in range(K)], axis=1)
    h1_cat = lax.dot_general(rel_cat, wd1c_ref[...], nn,
                             preferred_element_type=jnp.float32)
    h1_cat = jnp.maximum(h1_cat + bd1t_ref[...], 0.0)

    prod_cat = jnp.concatenate([q * unpack_k(kv4[k]) for k in range(K)],
                               axis=1)
    logits = lax.dot_general(prod_cat, gcat_ref[...], nn,
                             preferred_element_type=jnp.float32)
    l3 = logits[:, :K * H].reshape(PD, H, K)
    m = jnp.max(l3, axis=2, keepdims=True)
    e = jnp.exp(l3 - m)
    ssum = jnp.sum(e, axis=2, keepdims=True)
    attn = (e / ssum).reshape(PD, K * H)
    wf_cat = lax.dot_general(attn, bcat_ref[...], nn,
                             preferred_element_type=jnp.float32)

    vsum = jnp.zeros((PD, DM), jnp.float32)
    asum = jnp.zeros((PD, DM), jnp.float32)
    for k in range(K):
        wf = wf_cat[:, k * DM:(k + 1) * DM]
        vsum = vsum + wf * unpack_v(kv4[k])
        asum = asum + wf * h1_cat[:, k * DM:(k + 1) * DM]
    attn_out = vsum + lax.dot_general(asum, wd2_ref[...], nt,
                                      preferred_element_type=jnp.float32)
    attn_out = attn_out + bd2_ref[...]

    g = lax.dot_general(attn_out, wg1_ref[...], nt,
                        preferred_element_type=jnp.float32)
    g = jnp.maximum(g + bg1_ref[...], 0.0)
    g = lax.dot_general(g, wg2_ref[...], nt,
                        preferred_element_type=jnp.float32) + bg2_ref[...]
    res = lax.dot_general(wfc2_ref[...], g, nt,
                          preferred_element_type=jnp.float32)
    out_ref[0] = res + beta2_ref[...] + x_ref[0]


def _attention(q, kvn, xyzn, xyzp, x, gcat, bcat, wd1c, bd1t, w_d2, b_d2,
               w_g1, b_g1, w_g2, b_g2, wfc2s, beta2):
    full = lambda a: pl.BlockSpec(a.shape, lambda b, n: (0,) * a.ndim)
    return pl.pallas_call(
        _attn_body,
        grid=(1, N // PD),
        in_specs=[
            pl.BlockSpec((1, PD, DM), lambda b, n: (b, n, 0)),
            pl.BlockSpec((1, K, PD, DM), lambda b, n: (b, 0, n, 0)),
            pl.BlockSpec((1, K, PD, 8), lambda b, n: (b, 0, n, 0)),
            pl.BlockSpec((1, PD, 8), lambda b, n: (b, n, 0)),
            pl.BlockSpec((1, DP, PD), lambda b, n: (b, 0, n)),
            full(gcat), full(bcat), full(wd1c), full(bd1t), full(w_d2),
            full(b_d2), full(w_g1), full(b_g1), full(w_g2), full(b_g2),
            full(wfc2s), full(beta2),
        ],
        out_specs=pl.BlockSpec((1, DP, PD), lambda b, n: (b, 0, n)),
        out_shape=jax.ShapeDtypeStruct((1, DP, N), jnp.float32),
    )(q, kvn, xyzn, xyzp, x, gcat, bcat, wd1c, bd1t, w_d2, b_d2, w_g1, b_g1,
      w_g2, b_g2, wfc2s, beta2)


def kernel(x, xyz, w_fc1, b_fc1, bn1_g, bn1_b, mem_decay, thr_adapt,
           refr_decay, thr_base, w_q, w_k, w_v, w_d1, b_d1, w_d2, b_d2,
           w_g1, b_g1, w_g2, b_g2, w_fc2, b_fc2, bn2_g, bn2_b):
    xyzp = jnp.concatenate(
        [xyz, jnp.zeros((B, N, 5), jnp.float32)], axis=-1)

    alpha1 = (bn1_g * BN_SCALE).reshape(1, DM)
    beta1 = (b_fc1 * bn1_g * BN_SCALE + bn1_b).reshape(1, DM)
    mdp = mem_decay.reshape(1, DM)
    tap = thr_adapt.reshape(1, DM)
    rdp = refr_decay.reshape(1, DM)
    tbp = thr_base.reshape(1, DM)

    wd1p = jnp.concatenate([w_d1, jnp.zeros((DM, 5), jnp.float32)], axis=-1)
    wd1c = jnp.kron(jnp.eye(K, dtype=jnp.float32), wd1p.T)
    bd1t = jnp.tile(b_d1.reshape(1, DM), (1, K))
    h_of_m = jnp.arange(DM, dtype=jnp.int32) // HD
    g3 = jax.nn.one_hot(K * h_of_m[None, :]
                        + jnp.arange(K, dtype=jnp.int32)[:, None], 128,
                        dtype=jnp.float32)
    ind = g3.reshape(K * DM, 128)
    gcat = ind * (1.0 / (HD ** 0.5))
    bcat = ind[:, :K * H].T

    alpha2 = bn2_g * BN_SCALE
    wfc2s = w_fc2 * alpha2[:, None]
    beta2 = (b_fc2 * alpha2 + bn2_b).reshape(DP, 1)

    outs = []
    for b in range(B):
        xyzp_b = lax.slice_in_dim(xyzp, b, b + 1, axis=0)
        x_b = lax.slice_in_dim(x, b, b + 1, axis=0)
        idx_b, q_b, kv_b = _features(xyzp_b, x_b, w_fc1, alpha1, beta1,
                                     mdp, tap, rdp, tbp, w_q, w_k, w_v)
        idxg_b = idx_b.reshape(N, K).transpose(1, 0).reshape(N * K)
        kvn_b, xyzn_b = _sc_gather(
            kv_b.reshape(N, DM), xyzp_b.reshape(N, 8), idxg_b)
        outs.append(_attention(
            q_b, kvn_b.reshape(1, K, N, DM), xyzn_b.reshape(1, K, N, 8),
            xyzp_b, x_b, gcat, bcat, wd1c, bd1t, w_d2, b_d2.reshape(1, DM),
            w_g1, b_g1.reshape(1, DM), w_g2, b_g2.reshape(1, DM),
            wfc2s, beta2))
    return jnp.concatenate(outs, axis=0)


# speedup vs baseline: 17.1004x; 1.0110x over previous
import functools

import jax
import jax.numpy as jnp
from jax import lax
from jax.experimental import pallas as pl
from jax.experimental.pallas import tpu as pltpu
from jax.experimental.pallas import tpu_sc as plsc

B, N, DP, DM, K, H, T = 4, 2048, 128, 256, 16, 4, 4
HD = DM // H
INV_SQRT_2PI = 0.3989422804014327
BN_SCALE = 1.0 / (1.0 + 1e-5) ** 0.5

PA = 256


PB = 256


def _sigmoid(v):
    return 1.0 / (1.0 + jnp.exp(-v))


def _spike_fn(v):
    vc = jnp.clip(v, -10.0, 10.0)
    return 0.5 * jnp.exp(-vc * vc * 0.5) * INV_SQRT_2PI + 0.5 * _sigmoid(10.0 * vc)


def _lif_rows(x, mdv, tav, rdv, tb):
    mem = jnp.zeros_like(x)
    thr = jnp.broadcast_to(tb, x.shape)
    refr = jnp.zeros_like(x)
    acc = jnp.zeros_like(x)
    for _ in range(T):
        xi = x * (refr <= 0.0).astype(x.dtype)
        mem = mem * mdv * (1.0 - refr) + xi
        sp = _spike_fn(mem - thr)
        mem = mem * (1.0 - sp)
        refr = refr * rdv + sp
        thr = thr + tav * sp
        thr = tb + (thr - tb) * 0.95
        acc = acc + sp
    return acc * (1.0 / T)


def _feat_body(xt_ref, xf_ref, x_ref, wfc1_ref, alpha_ref, beta_ref,
               md_ref, ta_ref, rd_ref, tb_ref, wq_ref, wk_ref, wv_ref,
               idx_ref, q_ref, kv_ref):
    xt = xt_ref[0]
    xf = xf_ref[0]
    s = lax.dot_general(2.0 * xt, xf, (((1,), (1,)), ((), ())),
                        preferred_element_type=jnp.float32)
    ones = jnp.ones((1, 8), jnp.float32)
    xx = lax.dot_general(ones, xf * xf, (((1,), (1,)), ((), ())),
                         preferred_element_type=jnp.float32)
    s = s - xx
    iota = lax.broadcasted_iota(jnp.int32, (1, N), 1)
    cols = []
    for _ in range(K):
        m = jnp.max(s, axis=1, keepdims=True)
        msk = s >= m
        cand = jnp.where(msk, iota, N)
        a = jnp.min(cand, axis=1, keepdims=True)
        s = jnp.where(msk, -3.0e38, s)
        cols.append(a)
    idx_ref[0] = jnp.concatenate(cols, axis=1)

    xb = x_ref[0]
    pre = lax.dot_general(xb, wfc1_ref[...], (((0,), (1,)), ((), ())),
                          preferred_element_type=jnp.float32)
    pre = pre * alpha_ref[...] + beta_ref[...]
    mdv = jnp.clip(md_ref[...], 0.1, 0.99)
    tav = jnp.clip(ta_ref[...], 0.001, 0.1)
    rdv = jnp.clip(rd_ref[...], 0.1, 0.95)
    f = _lif_rows(pre, mdv, tav, rdv, tb_ref[...])
    nt = (((1,), (1,)), ((), ()))
    q_ref[0] = lax.dot_general(f, wq_ref[...], nt,
                               preferred_element_type=jnp.float32)
    kf = lax.dot_general(f, wk_ref[...], nt,
                         preferred_element_type=jnp.float32)
    vf = lax.dot_general(f, wv_ref[...], nt,
                         preferred_element_type=jnp.float32)
    k16 = lax.bitcast_convert_type(kf.astype(jnp.bfloat16), jnp.uint16)
    v16 = lax.bitcast_convert_type(vf.astype(jnp.bfloat16), jnp.uint16)
    packed = (v16.astype(jnp.uint32) << 16) | k16.astype(jnp.uint32)
    kv_ref[0] = lax.bitcast_convert_type(packed, jnp.int32)


def _features(xyzp, x, w_fc1, alpha1, beta1, md, ta, rd, tb, w_q, w_k, w_v):
    full = lambda a: pl.BlockSpec(a.shape, lambda b, n: (0,) * a.ndim)
    return pl.pallas_call(
        _feat_body,
        grid=(1, N // PB),
        in_specs=[
            pl.BlockSpec((1, PB, 8), lambda b, n: (b, n, 0)),
            pl.BlockSpec((1, N, 8), lambda b, n: (b, 0, 0)),
            pl.BlockSpec((1, DP, PB), lambda b, n: (b, 0, n)),
            full(w_fc1), full(alpha1), full(beta1), full(md), full(ta),
            full(rd), full(tb), full(w_q), full(w_k), full(w_v),
        ],
        out_specs=[
            pl.BlockSpec((1, PB, K), lambda b, n: (b, n, 0)),
            pl.BlockSpec((1, PB, DM), lambda b, n: (b, n, 0)),
            pl.BlockSpec((1, PB, DM), lambda b, n: (b, n, 0)),
        ],
        out_shape=[
            jax.ShapeDtypeStruct((1, N, K), jnp.int32),
            jax.ShapeDtypeStruct((1, N, DM), jnp.float32),
            jax.ShapeDtypeStruct((1, N, DM), jnp.int32),
        ],
    )(xyzp, xyzp, x, w_fc1, alpha1, beta1, md, ta, rd, tb, w_q, w_k, w_v)


NW = 32
CH = 128


def _sc_gather_fused(kv_flat, xyzp_vec, idxg):
    rows = idxg.shape[0]
    rpw = rows // NW
    nch = rpw // CH
    mesh = plsc.VectorSubcoreMesh(core_axis_name="c", subcore_axis_name="s")

    @functools.partial(
        pl.kernel, mesh=mesh,
        out_type=[
            jax.ShapeDtypeStruct((rows, DM), jnp.int32),
            jax.ShapeDtypeStruct((rows * 8,), jnp.float32),
        ],
        scratch_types=[
            pltpu.VMEM((N * 8,), jnp.float32),
            pltpu.VMEM((CH,), jnp.int32),
            pltpu.VMEM((CH,), jnp.int32),
            pltpu.VMEM((CH, DM), jnp.int32),
            pltpu.VMEM((CH, DM), jnp.int32),
            pltpu.VMEM((CH * 8,), jnp.float32),
            pltpu.VMEM((CH * 8,), jnp.float32),
            pltpu.SemaphoreType.DMA,
            pltpu.SemaphoreType.DMA,
        ],
        compiler_params=pltpu.CompilerParams(needs_layout_passes=False),
    )
    def gather(kv_hbm, xyzp_hbm, idx_hbm, kvn_hbm, xyzn_hbm,
               tab_v, idx_v0, idx_v1, kv_v0, kv_v1, out_v0, out_v1,
               sem0, sem1):
        wid = lax.axis_index("s") * 2 + lax.axis_index("c")
        base = wid * rpw
        pltpu.sync_copy(xyzp_hbm, tab_v)
        lane8 = lax.iota(jnp.int32, 16) * 8
        idx_bufs = [idx_v0, idx_v1]
        kv_bufs = [kv_v0, kv_v1]
        out_bufs = [out_v0, out_v1]
        sems = [sem0, sem1]

        def start(c):
            p = c % 2
            off = base + c * CH
            pltpu.sync_copy(idx_hbm.at[pl.ds(off, CH)], idx_bufs[p])
            return pltpu.async_copy(kv_hbm.at[idx_bufs[p]], kv_bufs[p],
                                    sems[p])

        cps = {0: start(0)}
        for c in range(nch):
            p = c % 2
            off = base + c * CH
            if c + 1 < nch:
                cps[c + 1] = start(c + 1)
            out_v = out_bufs[p]
            idx_v = idx_bufs[p]
            for jj in range(CH // 16):
                rbase = idx_v[pl.ds(jj * 16, 16)] * 8
                for cc in range(8):
                    vals = plsc.load_gather(tab_v, [rbase + cc])
                    plsc.store_scatter(out_v, [lane8 + (jj * 128 + cc)],
                                       vals)
            pltpu.sync_copy(out_v, xyzn_hbm.at[pl.ds(off * 8, CH * 8)])
            cps[c].wait()
            pltpu.sync_copy(kv_bufs[p], kvn_hbm.at[pl.ds(off, CH)])

    return gather(kv_flat, xyzp_vec, idxg)


def _sc_gather(kv_flat, xyzp_flat, idxg):
    kvn, xyzn = _sc_gather_fused(kv_flat, xyzp_flat.reshape(-1), idxg)
    return kvn, xyzn.reshape(-1, 8)


PD = 128
S = PD * K


def _attn_body(q_ref, kvn_ref, xyzn_ref, xq_ref, x_ref, gcat_ref, bcat_ref,
               wd1c_ref, bd1t_ref, wd2_ref, bd2_ref, wg1_ref, bg1_ref,
               wg2_ref, bg2_ref, wfc2_ref, beta2_ref, out_ref):
    kv4 = lax.bitcast_convert_type(kvn_ref[0], jnp.uint32)
    xyzn4 = xyzn_ref[0]
    xq = xq_ref[0]
    q = q_ref[0]
    nt = (((1,), (1,)), ((), ()))
    nn = (((1,), (0,)), ((), ()))

    def unpack_k(u):
        return lax.bitcast_convert_type(u.astype(jnp.uint16), jnp.bfloat16)

    def unpack_v(u):
        return lax.bitcast_convert_type((u >> 16).astype(jnp.uint16),
                                        jnp.bfloat16)

    rel_cat = jnp.concatenate([xq - xyzn4[k] for k in range(K)],
                              axis=1).astype(jnp.bfloat16)
    h1_cat = lax.dot_general(rel_cat, wd1c_ref[...], nn,
                             preferred_element_type=jnp.float32)
    h1_cat = jnp.maximum(h1_cat + bd1t_ref[...], 0.0).astype(jnp.bfloat16)

    qb = q.astype(jnp.bfloat16)
    prod_cat = jnp.concatenate([qb * unpack_k(kv4[k]) for k in range(K)],
                               axis=1)
    logits = lax.dot_general(prod_cat, gcat_ref[...], nn,
                             preferred_element_type=jnp.float32)
    l3 = logits[:, :K * H].reshape(PD, H, K)
    m = jnp.max(l3, axis=2, keepdims=True)
    e = jnp.exp(l3 - m)
    ssum = jnp.sum(e, axis=2, keepdims=True)
    attn = (e / ssum).reshape(PD, K * H).astype(jnp.bfloat16)
    wf_cat = lax.dot_general(attn, bcat_ref[...], nn,
                             preferred_element_type=jnp.float32
                             ).astype(jnp.bfloat16)

    vsum = jnp.zeros((PD, DM), jnp.float32)
    asum = jnp.zeros((PD, DM), jnp.float32)
    for k in range(K):
        wf = wf_cat[:, k * DM:(k + 1) * DM]
        vsum = vsum + (wf * unpack_v(kv4[k])).astype(jnp.float32)
        asum = asum + (wf * h1_cat[:, k * DM:(k + 1) * DM]).astype(
            jnp.float32)
    attn_out = vsum + lax.dot_general(asum, wd2_ref[...], nt,
                                      preferred_element_type=jnp.float32)
    attn_out = attn_out + bd2_ref[...]

    g = lax.dot_general(attn_out, wg1_ref[...], nt,
                        preferred_element_type=jnp.float32)
    g = jnp.maximum(g + bg1_ref[...], 0.0)
    g = lax.dot_general(g, wg2_ref[...], nt,
                        preferred_element_type=jnp.float32) + bg2_ref[...]
    res = lax.dot_general(wfc2_ref[...], g, nt,
                          preferred_element_type=jnp.float32)
    out_ref[0] = res + beta2_ref[...] + x_ref[0]


def _attention(q, kvn, xyzn, xyzp, x, gcat, bcat, wd1c, bd1t, w_d2, b_d2,
               w_g1, b_g1, w_g2, b_g2, wfc2s, beta2):
    full = lambda a: pl.BlockSpec(a.shape, lambda b, n: (0,) * a.ndim)
    return pl.pallas_call(
        _attn_body,
        grid=(1, N // PD),
        in_specs=[
            pl.BlockSpec((1, PD, DM), lambda b, n: (b, n, 0)),
            pl.BlockSpec((1, K, PD, DM), lambda b, n: (b, 0, n, 0)),
            pl.BlockSpec((1, K, PD, 8), lambda b, n: (b, 0, n, 0)),
            pl.BlockSpec((1, PD, 8), lambda b, n: (b, n, 0)),
            pl.BlockSpec((1, DP, PD), lambda b, n: (b, 0, n)),
            full(gcat), full(bcat), full(wd1c), full(bd1t), full(w_d2),
            full(b_d2), full(w_g1), full(b_g1), full(w_g2), full(b_g2),
            full(wfc2s), full(beta2),
        ],
        out_specs=pl.BlockSpec((1, DP, PD), lambda b, n: (b, 0, n)),
        out_shape=jax.ShapeDtypeStruct((1, DP, N), jnp.float32),
    )(q, kvn, xyzn, xyzp, x, gcat, bcat, wd1c, bd1t, w_d2, b_d2, w_g1, b_g1,
      w_g2, b_g2, wfc2s, beta2)


def kernel(x, xyz, w_fc1, b_fc1, bn1_g, bn1_b, mem_decay, thr_adapt,
           refr_decay, thr_base, w_q, w_k, w_v, w_d1, b_d1, w_d2, b_d2,
           w_g1, b_g1, w_g2, b_g2, w_fc2, b_fc2, bn2_g, bn2_b):
    xyzp = jnp.concatenate(
        [xyz, jnp.zeros((B, N, 5), jnp.float32)], axis=-1)

    alpha1 = (bn1_g * BN_SCALE).reshape(1, DM)
    beta1 = (b_fc1 * bn1_g * BN_SCALE + bn1_b).reshape(1, DM)
    mdp = mem_decay.reshape(1, DM)
    tap = thr_adapt.reshape(1, DM)
    rdp = refr_decay.reshape(1, DM)
    tbp = thr_base.reshape(1, DM)

    wd1p = jnp.concatenate([w_d1, jnp.zeros((DM, 5), jnp.float32)], axis=-1)
    wd1c = jnp.kron(jnp.eye(K, dtype=jnp.float32), wd1p.T).astype(
        jnp.bfloat16)
    bd1t = jnp.tile(b_d1.reshape(1, DM), (1, K))
    h_of_m = jnp.arange(DM, dtype=jnp.int32) // HD
    g3 = jax.nn.one_hot(K * h_of_m[None, :]
                        + jnp.arange(K, dtype=jnp.int32)[:, None], 128,
                        dtype=jnp.float32)
    ind = g3.reshape(K * DM, 128)
    gcat = (ind * (1.0 / (HD ** 0.5))).astype(jnp.bfloat16)
    bcat = ind[:, :K * H].T.astype(jnp.bfloat16)

    alpha2 = bn2_g * BN_SCALE
    wfc2s = w_fc2 * alpha2[:, None]
    beta2 = (b_fc2 * alpha2 + bn2_b).reshape(DP, 1)

    outs = []
    for b in range(B):
        xyzp_b = lax.slice_in_dim(xyzp, b, b + 1, axis=0)
        x_b = lax.slice_in_dim(x, b, b + 1, axis=0)
        idx_b, q_b, kv_b = _features(xyzp_b, x_b, w_fc1, alpha1, beta1,
                                     mdp, tap, rdp, tbp, w_q, w_k, w_v)
        idxg_b = idx_b.reshape(N, K).transpose(1, 0).reshape(N * K)
        kvn_b, xyzn_b = _sc_gather(
            kv_b.reshape(N, DM), xyzp_b.reshape(N, 8), idxg_b)
        outs.append(_attention(
            q_b, kvn_b.reshape(1, K, N, DM), xyzn_b.reshape(1, K, N, 8),
            xyzp_b, x_b, gcat, bcat, wd1c, bd1t, w_d2, b_d2.reshape(1, DM),
            w_g1, b_g1.reshape(1, DM), w_g2, b_g2.reshape(1, DM),
            wfc2s, beta2))
    return jnp.concatenate(outs, axis=0)


# speedup vs baseline: 17.8295x; 1.0426x over previous
import functools

import jax
import jax.numpy as jnp
from jax import lax
from jax.experimental import pallas as pl
from jax.experimental.pallas import tpu as pltpu
from jax.experimental.pallas import tpu_sc as plsc

B, N, DP, DM, K, H, T = 4, 2048, 128, 256, 16, 4, 4
HD = DM // H
INV_SQRT_2PI = 0.3989422804014327
BN_SCALE = 1.0 / (1.0 + 1e-5) ** 0.5

PA = 256


PB = 256


def _sigmoid(v):
    return 1.0 / (1.0 + jnp.exp(-v))


def _spike_fn(v):
    vc = jnp.clip(v, -10.0, 10.0)
    return 0.5 * jnp.exp(-vc * vc * 0.5) * INV_SQRT_2PI + 0.5 * _sigmoid(10.0 * vc)


def _lif_rows(x, mdv, tav, rdv, tb):
    mem = jnp.zeros_like(x)
    thr = jnp.broadcast_to(tb, x.shape)
    refr = jnp.zeros_like(x)
    acc = jnp.zeros_like(x)
    for _ in range(T):
        xi = x * (refr <= 0.0).astype(x.dtype)
        mem = mem * mdv * (1.0 - refr) + xi
        sp = _spike_fn(mem - thr)
        mem = mem * (1.0 - sp)
        refr = refr * rdv + sp
        thr = thr + tav * sp
        thr = tb + (thr - tb) * 0.95
        acc = acc + sp
    return acc * (1.0 / T)


def _feat_body(xt_ref, xf_ref, x_ref, wfc1_ref, alpha_ref, beta_ref,
               md_ref, ta_ref, rd_ref, tb_ref, wq_ref, wk_ref, wv_ref,
               idx_ref, q_ref, kv_ref):
    xt = xt_ref[0]
    xf = xf_ref[0]
    s = lax.dot_general(2.0 * xt, xf, (((1,), (1,)), ((), ())),
                        preferred_element_type=jnp.float32)
    ones = jnp.ones((1, 8), jnp.float32)
    xx = lax.dot_general(ones, xf * xf, (((1,), (1,)), ((), ())),
                         preferred_element_type=jnp.float32)
    s = s - xx
    iota = lax.broadcasted_iota(jnp.int32, (1, N), 1)
    cols = []
    for _ in range(K):
        m = jnp.max(s, axis=1, keepdims=True)
        msk = s >= m
        cand = jnp.where(msk, iota, N)
        a = jnp.min(cand, axis=1, keepdims=True)
        s = jnp.where(msk, -3.0e38, s)
        cols.append(a)
    idx_ref[0] = jnp.concatenate(cols, axis=1)

    xb = x_ref[0]
    pre = lax.dot_general(xb, wfc1_ref[...], (((0,), (1,)), ((), ())),
                          preferred_element_type=jnp.float32)
    pre = pre * alpha_ref[...] + beta_ref[...]
    mdv = jnp.clip(md_ref[...], 0.1, 0.99)
    tav = jnp.clip(ta_ref[...], 0.001, 0.1)
    rdv = jnp.clip(rd_ref[...], 0.1, 0.95)
    f = _lif_rows(pre, mdv, tav, rdv, tb_ref[...])
    nt = (((1,), (1,)), ((), ()))
    q_ref[0] = lax.dot_general(f, wq_ref[...], nt,
                               preferred_element_type=jnp.float32)
    kf = lax.dot_general(f, wk_ref[...], nt,
                         preferred_element_type=jnp.float32)
    vf = lax.dot_general(f, wv_ref[...], nt,
                         preferred_element_type=jnp.float32)
    k16 = lax.bitcast_convert_type(kf.astype(jnp.bfloat16), jnp.uint16)
    v16 = lax.bitcast_convert_type(vf.astype(jnp.bfloat16), jnp.uint16)
    packed = (v16.astype(jnp.uint32) << 16) | k16.astype(jnp.uint32)
    kv_ref[0] = lax.bitcast_convert_type(packed, jnp.int32)


def _features(xyzp, x, w_fc1, alpha1, beta1, md, ta, rd, tb, w_q, w_k, w_v):
    full = lambda a: pl.BlockSpec(a.shape, lambda b, n: (0,) * a.ndim)
    return pl.pallas_call(
        _feat_body,
        grid=(1, N // PB),
        in_specs=[
            pl.BlockSpec((1, PB, 8), lambda b, n: (b, n, 0)),
            pl.BlockSpec((1, N, 8), lambda b, n: (b, 0, 0)),
            pl.BlockSpec((1, DP, PB), lambda b, n: (b, 0, n)),
            full(w_fc1), full(alpha1), full(beta1), full(md), full(ta),
            full(rd), full(tb), full(w_q), full(w_k), full(w_v),
        ],
        out_specs=[
            pl.BlockSpec((1, PB, K), lambda b, n: (b, n, 0)),
            pl.BlockSpec((1, PB, DM), lambda b, n: (b, n, 0)),
            pl.BlockSpec((1, PB, DM), lambda b, n: (b, n, 0)),
        ],
        out_shape=[
            jax.ShapeDtypeStruct((1, N, K), jnp.int32),
            jax.ShapeDtypeStruct((1, N, DM), jnp.float32),
            jax.ShapeDtypeStruct((1, N, DM), jnp.int32),
        ],
    )(xyzp, xyzp, x, w_fc1, alpha1, beta1, md, ta, rd, tb, w_q, w_k, w_v)


NW = 32
CH = 128


def _sc_gather_fused(kv_flat, xyzp_vec, idxg):
    rows = idxg.shape[0]
    rpw = rows // NW
    nch = rpw // CH
    mesh = plsc.VectorSubcoreMesh(core_axis_name="c", subcore_axis_name="s")

    @functools.partial(
        pl.kernel, mesh=mesh,
        out_type=[
            jax.ShapeDtypeStruct((rows, DM), jnp.int32),
            jax.ShapeDtypeStruct((rows * 8,), jnp.float32),
        ],
        scratch_types=[
            pltpu.VMEM((N * 8,), jnp.float32),
            pltpu.VMEM((CH,), jnp.int32),
            pltpu.VMEM((CH,), jnp.int32),
            pltpu.VMEM((CH, DM), jnp.int32),
            pltpu.VMEM((CH, DM), jnp.int32),
            pltpu.VMEM((CH * 8,), jnp.float32),
            pltpu.VMEM((CH * 8,), jnp.float32),
            pltpu.SemaphoreType.DMA,
            pltpu.SemaphoreType.DMA,
        ],
        compiler_params=pltpu.CompilerParams(needs_layout_passes=False),
    )
    def gather(kv_hbm, xyzp_hbm, idx_hbm, kvn_hbm, xyzn_hbm,
               tab_v, idx_v0, idx_v1, kv_v0, kv_v1, out_v0, out_v1,
               sem0, sem1):
        wid = lax.axis_index("s") * 2 + lax.axis_index("c")
        base = wid * rpw
        pltpu.sync_copy(xyzp_hbm, tab_v)
        lane8 = lax.iota(jnp.int32, 16) * 8
        idx_bufs = [idx_v0, idx_v1]
        kv_bufs = [kv_v0, kv_v1]
        out_bufs = [out_v0, out_v1]
        sems = [sem0, sem1]

        def start(c):
            p = c % 2
            off = base + c * CH
            pltpu.sync_copy(idx_hbm.at[pl.ds(off, CH)], idx_bufs[p])
            return pltpu.async_copy(kv_hbm.at[idx_bufs[p]], kv_bufs[p],
                                    sems[p])

        cps = {0: start(0)}
        for c in range(nch):
            p = c % 2
            off = base + c * CH
            if c + 1 < nch:
                cps[c + 1] = start(c + 1)
            out_v = out_bufs[p]
            idx_v = idx_bufs[p]
            for jj in range(CH // 16):
                rbase = idx_v[pl.ds(jj * 16, 16)] * 8
                for cc in range(8):
                    vals = plsc.load_gather(tab_v, [rbase + cc])
                    plsc.store_scatter(out_v, [lane8 + (jj * 128 + cc)],
                                       vals)
            pltpu.sync_copy(out_v, xyzn_hbm.at[pl.ds(off * 8, CH * 8)])
            cps[c].wait()
            pltpu.sync_copy(kv_bufs[p], kvn_hbm.at[pl.ds(off, CH)])

    return gather(kv_flat, xyzp_vec, idxg)


def _sc_gather(kv_flat, xyzp_flat, idxg):
    kvn, xyzn = _sc_gather_fused(kv_flat, xyzp_flat.reshape(-1), idxg)
    return kvn, xyzn.reshape(-1, 8)


PD = 256
S = PD * K


def _attn_body(q_ref, kvn_ref, xyzn_ref, xq_ref, x_ref, gcat_ref, bcat_ref,
               wd1c_ref, bd1t_ref, wd2_ref, bd2_ref, wg1_ref, bg1_ref,
               wg2_ref, bg2_ref, wfc2_ref, beta2_ref, out_ref):
    kv4 = lax.bitcast_convert_type(kvn_ref[0], jnp.uint32)
    xyzn4 = xyzn_ref[0]
    xq = xq_ref[0]
    q = q_ref[0]
    nt = (((1,), (1,)), ((), ()))
    nn = (((1,), (0,)), ((), ()))

    def unpack_k(u):
        return lax.bitcast_convert_type(u.astype(jnp.uint16), jnp.bfloat16)

    def unpack_v(u):
        return lax.bitcast_convert_type((u >> 16).astype(jnp.uint16),
                                        jnp.bfloat16)

    rel_cat = jnp.concatenate([xq - xyzn4[k] for k in range(K)],
                              axis=1).astype(jnp.bfloat16)
    h1_cat = lax.dot_general(rel_cat, wd1c_ref[...], nn,
                             preferred_element_type=jnp.float32)
    h1_cat = jnp.maximum(h1_cat + bd1t_ref[...], 0.0).astype(jnp.bfloat16)

    qb = q.astype(jnp.bfloat16)
    prod_cat = jnp.concatenate([qb * unpack_k(kv4[k]) for k in range(K)],
                               axis=1)
    logits = lax.dot_general(prod_cat, gcat_ref[...], nn,
                             preferred_element_type=jnp.float32)
    l3 = logits[:, :K * H].reshape(PD, H, K)
    m = jnp.max(l3, axis=2, keepdims=True)
    e = jnp.exp(l3 - m)
    ssum = jnp.sum(e, axis=2, keepdims=True)
    attn = (e / ssum).reshape(PD, K * H).astype(jnp.bfloat16)
    wf_cat = lax.dot_general(attn, bcat_ref[...], nn,
                             preferred_element_type=jnp.float32
                             ).astype(jnp.bfloat16)

    vsum = jnp.zeros((PD, DM), jnp.float32)
    asum = jnp.zeros((PD, DM), jnp.float32)
    for k in range(K):
        wf = wf_cat[:, k * DM:(k + 1) * DM]
        vsum = vsum + (wf * unpack_v(kv4[k])).astype(jnp.float32)
        asum = asum + (wf * h1_cat[:, k * DM:(k + 1) * DM]).astype(
            jnp.float32)
    attn_out = vsum + lax.dot_general(asum, wd2_ref[...], nt,
                                      preferred_element_type=jnp.float32)
    attn_out = attn_out + bd2_ref[...]

    g = lax.dot_general(attn_out, wg1_ref[...], nt,
                        preferred_element_type=jnp.float32)
    g = jnp.maximum(g + bg1_ref[...], 0.0)
    g = lax.dot_general(g, wg2_ref[...], nt,
                        preferred_element_type=jnp.float32) + bg2_ref[...]
    res = lax.dot_general(wfc2_ref[...], g, nt,
                          preferred_element_type=jnp.float32)
    out_ref[0] = res + beta2_ref[...] + x_ref[0]


def _attention(q, kvn, xyzn, xyzp, x, gcat, bcat, wd1c, bd1t, w_d2, b_d2,
               w_g1, b_g1, w_g2, b_g2, wfc2s, beta2):
    full = lambda a: pl.BlockSpec(a.shape, lambda b, n: (0,) * a.ndim)
    return pl.pallas_call(
        _attn_body,
        grid=(1, N // PD),
        in_specs=[
            pl.BlockSpec((1, PD, DM), lambda b, n: (b, n, 0)),
            pl.BlockSpec((1, K, PD, DM), lambda b, n: (b, 0, n, 0)),
            pl.BlockSpec((1, K, PD, 8), lambda b, n: (b, 0, n, 0)),
            pl.BlockSpec((1, PD, 8), lambda b, n: (b, n, 0)),
            pl.BlockSpec((1, DP, PD), lambda b, n: (b, 0, n)),
            full(gcat), full(bcat), full(wd1c), full(bd1t), full(w_d2),
            full(b_d2), full(w_g1), full(b_g1), full(w_g2), full(b_g2),
            full(wfc2s), full(beta2),
        ],
        out_specs=pl.BlockSpec((1, DP, PD), lambda b, n: (b, 0, n)),
        out_shape=jax.ShapeDtypeStruct((1, DP, N), jnp.float32),
    )(q, kvn, xyzn, xyzp, x, gcat, bcat, wd1c, bd1t, w_d2, b_d2, w_g1, b_g1,
      w_g2, b_g2, wfc2s, beta2)


def kernel(x, xyz, w_fc1, b_fc1, bn1_g, bn1_b, mem_decay, thr_adapt,
           refr_decay, thr_base, w_q, w_k, w_v, w_d1, b_d1, w_d2, b_d2,
           w_g1, b_g1, w_g2, b_g2, w_fc2, b_fc2, bn2_g, bn2_b):
    xyzp = jnp.concatenate(
        [xyz, jnp.zeros((B, N, 5), jnp.float32)], axis=-1)

    alpha1 = (bn1_g * BN_SCALE).reshape(1, DM)
    beta1 = (b_fc1 * bn1_g * BN_SCALE + bn1_b).reshape(1, DM)
    mdp = mem_decay.reshape(1, DM)
    tap = thr_adapt.reshape(1, DM)
    rdp = refr_decay.reshape(1, DM)
    tbp = thr_base.reshape(1, DM)

    wd1p = jnp.concatenate([w_d1, jnp.zeros((DM, 5), jnp.float32)], axis=-1)
    wd1c = jnp.kron(jnp.eye(K, dtype=jnp.float32), wd1p.T).astype(
        jnp.bfloat16)
    bd1t = jnp.tile(b_d1.reshape(1, DM), (1, K))
    h_of_m = jnp.arange(DM, dtype=jnp.int32) // HD
    g3 = jax.nn.one_hot(K * h_of_m[None, :]
                        + jnp.arange(K, dtype=jnp.int32)[:, None], 128,
                        dtype=jnp.float32)
    ind = g3.reshape(K * DM, 128)
    gcat = (ind * (1.0 / (HD ** 0.5))).astype(jnp.bfloat16)
    bcat = ind[:, :K * H].T.astype(jnp.bfloat16)

    alpha2 = bn2_g * BN_SCALE
    wfc2s = w_fc2 * alpha2[:, None]
    beta2 = (b_fc2 * alpha2 + bn2_b).reshape(DP, 1)

    outs = []
    for b in range(B):
        xyzp_b = lax.slice_in_dim(xyzp, b, b + 1, axis=0)
        x_b = lax.slice_in_dim(x, b, b + 1, axis=0)
        idx_b, q_b, kv_b = _features(xyzp_b, x_b, w_fc1, alpha1, beta1,
                                     mdp, tap, rdp, tbp, w_q, w_k, w_v)
        idxg_b = idx_b.reshape(N, K).transpose(1, 0).reshape(N * K)
        kvn_b, xyzn_b = _sc_gather(
            kv_b.reshape(N, DM), xyzp_b.reshape(N, 8), idxg_b)
        outs.append(_attention(
            q_b, kvn_b.reshape(1, K, N, DM), xyzn_b.reshape(1, K, N, 8),
            xyzp_b, x_b, gcat, bcat, wd1c, bd1t, w_d2, b_d2.reshape(1, DM),
            w_g1, b_g1.reshape(1, DM), w_g2, b_g2.reshape(1, DM),
            wfc2s, beta2))
    return jnp.concatenate(outs, axis=0)
